# baseline (speedup 1.0000x reference)
import functools

import jax
import jax.numpy as jnp
from jax import lax
from jax.experimental import pallas as pl
from jax.experimental.pallas import tpu as pltpu
from jax.experimental.pallas import tpu_sc as plsc

_N_HOST = 10000
_N_FLOW = 50000
_D = 128
_E = 160000

_NCORE = 2
_NSUB = 16
_LANES = 16

_EP = _E // _NSUB
_SCAN_ROWS = _EP // _LANES
_BATCH = 128
_NB_MAX = _EP // _BATCH

_SC_PARAMS = dict(
    compiler_params=pltpu.CompilerParams(needs_layout_passes=False,
                                         use_tc_tiling_on_sc=False))


def _sc_mesh():
  return plsc.VectorSubcoreMesh(core_axis_name="c", subcore_axis_name="s",
                                num_cores=_NCORE, num_subcores=_NSUB)


def _zero_rowbuf(rowbuf):
  z16 = jnp.zeros((_LANES,), jnp.float32)

  def zb(i, _):
    for k in range(_D // _LANES):
      rowbuf[i, pl.ds(k * _LANES, _LANES)] = z16
    return 0
  lax.fori_loop(0, _BATCH, zb, 0)


def _compact_chunk(ev, dstbuf, srcbuf, lo, ch, dump):
  iota = jnp.arange(_LANES, dtype=jnp.int32)
  zi16 = jnp.zeros((_LANES,), jnp.int32)

  def scan_body(j, posv):
    p16 = ev[j]
    d16 = lax.shift_right_logical(p16, jnp.full((_LANES,), 16, jnp.int32))
    inm = (d16 >= lo) & (d16 < lo + ch)
    ex = plsc.cumsum(inm.astype(jnp.int32))
    tgt = posv + ex - 1
    row = jnp.right_shift(tgt, 7)
    col = jnp.bitwise_and(tgt, _BATCH - 1)
    plsc.store_scatter(dstbuf, [row, col], d16 - lo, mask=inm)
    if srcbuf is not None:
      plsc.store_scatter(srcbuf, [row, col],
                         jnp.bitwise_and(p16, 0xFFFF), mask=inm)
    return posv + plsc.all_reduce_population_count(inm)
  posv = lax.fori_loop(0, _SCAN_ROWS, scan_body, zi16)

  nbv = jnp.right_shift(posv + (_BATCH - 1), 7)
  lastrow = nbv - 1
  for k in range(_BATCH // _LANES):
    colk = k * _LANES + iota
    flatp = lastrow * _BATCH + colk
    m = flatp >= posv
    plsc.store_scatter(dstbuf, [lastrow, colk],
                       jnp.full((_LANES,), dump, jnp.int32), mask=m)
    if srcbuf is not None:
      plsc.store_scatter(srcbuf, [lastrow, colk], zi16, mask=m)
  return jnp.max(nbv)


_CH_FLOW = 6400
_CH_HOST = 5120
_NPAD_FLOW = 8 * _CH_FLOW
_NPAD_HOST = 2 * _CH_HOST
_CHC_FLOW = 25008
_CHC_HOST = 5008


def _agg_relation(x_hbm, e_hbm, agg_hbm, ch, n_dst, refs):
  (ev, srcbuf, dstbuf, bufs, gsems, agg_s) = refs
  cid = lax.axis_index("c")
  sid = lax.axis_index("s")
  nchunk = -(-n_dst // ch)
  assert nchunk % _NCORE == 0 and ch % _NSUB == 0
  dump = ch
  rps = ch // _NSUB
  assert rps % 8 == 0

  pltpu.sync_copy(e_hbm.at[sid], ev)

  for p in range(nchunk // _NCORE):
    chunk = cid + _NCORE * p
    lo = chunk * ch

    _zero_rowbuf(bufs[0])
    rem = rps % _BATCH
    for k in range(rps // _BATCH):
      pltpu.sync_copy(bufs[0], agg_s.at[pl.ds(sid * rps + k * _BATCH,
                                              _BATCH)])
    if rem:
      pltpu.sync_copy(
          bufs[0].at[pl.ds(0, rem)],
          agg_s.at[pl.ds(sid * rps + (rps // _BATCH) * _BATCH, rem)])
    plsc.subcore_barrier()

    nb = _compact_chunk(ev, dstbuf, srcbuf, lo, ch, dump)

    for q in range(3):
      @pl.when(q < nb)
      def _(q=q):
        pltpu.async_copy(x_hbm.at[srcbuf.at[q]], bufs[q], gsems[q])

    def bat(g, _):
      for q in range(3):
        b = 3 * g + q

        @pl.when(b < nb)
        def _(b=b, q=q):
          pltpu.make_async_copy(x_hbm.at[srcbuf.at[b]], bufs[q],
                                gsems[q]).wait()
          pltpu.sync_copy(bufs[q], agg_s.at[dstbuf.at[b]], add=True)

          @pl.when(b + 3 < nb)
          def _():
            pltpu.async_copy(x_hbm.at[srcbuf.at[b + 3]], bufs[q],
                             gsems[q])
      return 0
    lax.fori_loop(0, (_NB_MAX + 2) // 3, bat, 0)

    plsc.subcore_barrier()

    base = lo + sid * rps
    for k in range(rps // _BATCH):
      pltpu.sync_copy(agg_s.at[pl.ds(sid * rps + k * _BATCH, _BATCH)],
                      agg_hbm.at[pl.ds(base + k * _BATCH, _BATCH)])
    if rem:
      pltpu.sync_copy(
          agg_s.at[pl.ds(sid * rps + (rps // _BATCH) * _BATCH, rem)],
          agg_hbm.at[pl.ds(base + (rps // _BATCH) * _BATCH, rem)])
    plsc.subcore_barrier()


def _make_layer_kernel(name: str):
  out_type = (
      jax.ShapeDtypeStruct((_NPAD_FLOW, _D), jnp.float32),
      jax.ShapeDtypeStruct((_NPAD_FLOW, _D), jnp.float32),
      jax.ShapeDtypeStruct((_NPAD_HOST, _D), jnp.float32),
      jax.ShapeDtypeStruct((_NPAD_HOST, _D), jnp.float32),
  )
  scratch = dict(
      ev=pltpu.VMEM((_SCAN_ROWS, _LANES), jnp.int32),
      srcbuf=pltpu.VMEM((_NB_MAX, _BATCH), jnp.int32),
      dstbuf=pltpu.VMEM((_NB_MAX, _BATCH), jnp.int32),
      rowbuf0=pltpu.VMEM((_BATCH, _D), jnp.float32),
      rowbuf1=pltpu.VMEM((_BATCH, _D), jnp.float32),
      rowbuf2=pltpu.VMEM((_BATCH, _D), jnp.float32),
      agg_s=pltpu.VMEM_SHARED((_CH_FLOW + 16, _D), jnp.float32),
      gsem0=pltpu.SemaphoreType.DMA,
      gsem1=pltpu.SemaphoreType.DMA,
      gsem2=pltpu.SemaphoreType.DMA,
  )

  def body(xh_hbm, xf_hbm, eS, eP, eR, eH, aS, aP, aR, aH, *, ev, srcbuf,
           dstbuf, rowbuf0, rowbuf1, rowbuf2, agg_s, gsem0, gsem1, gsem2):
    refs = (ev, srcbuf, dstbuf, (rowbuf0, rowbuf1, rowbuf2),
            (gsem0, gsem1, gsem2), agg_s)
    _agg_relation(xh_hbm, eS, aS, _CH_FLOW, _N_FLOW, refs)
    _agg_relation(xf_hbm, eP, aP, _CH_FLOW, _N_FLOW, refs)
    _agg_relation(xf_hbm, eR, aR, _CH_HOST, _N_HOST, refs)
    _agg_relation(xf_hbm, eH, aH, _CH_HOST, _N_HOST, refs)

  return pl.kernel(body, out_type=out_type, mesh=_sc_mesh(),
                   scratch_types=scratch, name=name, **_SC_PARAMS)


def _make_cnt_kernel(name: str):
  scratch = dict(
      ev=pltpu.VMEM((_SCAN_ROWS, _LANES), jnp.int32),
      dstbuf=pltpu.VMEM((_NB_MAX, _BATCH), jnp.int32),
      onesb=pltpu.VMEM((_BATCH, 16), jnp.float32),
      zc=pltpu.VMEM((_BATCH, 16), jnp.float32),
      cnt_s=pltpu.VMEM_SHARED((_CHC_FLOW + 16, 16), jnp.float32),
      sem=pltpu.SemaphoreType.DMA,
  )

  def body(eS, eP, eR, eH, cS, cP, cR, cH, *, ev, dstbuf, onesb, zc,
           cnt_s, sem):
    cid = lax.axis_index("c")
    sid = lax.axis_index("s")
    iota = jnp.arange(_LANES, dtype=jnp.int32)
    one0 = (iota == 0).astype(jnp.float32)
    z16 = jnp.zeros((_LANES,), jnp.float32)

    def ob(i, _):
      onesb[i, pl.ds(0, _LANES)] = one0
      zc[i, pl.ds(0, _LANES)] = z16
      return 0
    lax.fori_loop(0, _BATCH, ob, 0)

    for e_hbm, c_hbm, ch in ((eS, cS, _CHC_FLOW), (eP, cP, _CHC_FLOW),
                             (eR, cR, _CHC_HOST), (eH, cH, _CHC_HOST)):
      rps = ch // _NSUB
      dump = ch
      lo = cid * ch
      pltpu.sync_copy(e_hbm.at[sid], ev)

      for k in range(rps // _BATCH):
        pltpu.sync_copy(zc, cnt_s.at[pl.ds(sid * rps + k * _BATCH, _BATCH)])
      rem = rps % _BATCH
      if rem:
        pltpu.sync_copy(
            zc.at[pl.ds(0, rem)],
            cnt_s.at[pl.ds(sid * rps + (rps // _BATCH) * _BATCH, rem)])
      plsc.subcore_barrier()

      nb = _compact_chunk(ev, dstbuf, None, lo, ch, dump)

      def fire(b, _):
        @pl.when(b < nb)
        def _():
          pltpu.async_copy(onesb, cnt_s.at[dstbuf.at[b]], sem, add=True)
        return 0
      lax.fori_loop(0, _NB_MAX, fire, 0)

      def drain(b, _):
        @pl.when(b < nb)
        def _():
          pltpu.make_async_copy(onesb, cnt_s.at[dstbuf.at[b]], sem).wait()
        return 0
      lax.fori_loop(0, _NB_MAX, drain, 0)

      plsc.subcore_barrier()

      base = lo + sid * rps
      pltpu.sync_copy(cnt_s.at[pl.ds(sid * rps, rps)],
                      c_hbm.at[pl.ds(base, rps)])
      plsc.subcore_barrier()

  return pl.kernel(
      body,
      out_type=(jax.ShapeDtypeStruct((_NCORE * _CHC_FLOW, 16), jnp.float32),
                jax.ShapeDtypeStruct((_NCORE * _CHC_FLOW, 16), jnp.float32),
                jax.ShapeDtypeStruct((_NCORE * _CHC_HOST, 16), jnp.float32),
                jax.ShapeDtypeStruct((_NCORE * _CHC_HOST, 16), jnp.float32)),
      mesh=_sc_mesh(), scratch_types=scratch, name=name, **_SC_PARAMS)


@functools.cache
def _layerk(name):
  return _make_layer_kernel(name)


@functools.cache
def _cntk(name):
  return _make_cnt_kernel(name)


def _combine_stats_call(n, name):
  R = 1000
  grid = n // R

  def body(aggA, aggB, cA, cB, wA, wB, wr, bc, x, p_ref, st_ref, acc):
    i = pl.program_id(0)
    mA = aggA[...] / jnp.maximum(cA[...], 1.0)
    mB = aggB[...] / jnp.maximum(cB[...], 1.0)
    p = (jnp.dot(mA, wA[...], preferred_element_type=jnp.float32)
         + jnp.dot(mB, wB[...], preferred_element_type=jnp.float32)
         + jnp.dot(x[...], wr[...], preferred_element_type=jnp.float32)
         + bc[...])
    p_ref[...] = p
    s = jnp.sum(p, axis=0, keepdims=True)
    sq = jnp.sum(p * p, axis=0, keepdims=True)

    @pl.when(i == 0)
    def _():
      acc[...] = jnp.zeros_like(acc)

    acc[0:1, :] += s
    acc[1:2, :] += sq

    @pl.when(i == grid - 1)
    def _():
      st_ref[...] = acc[...]

  return pl.pallas_call(
      body,
      grid=(grid,),
      in_specs=[
          pl.BlockSpec((R, _D), lambda i: (i, 0)),
          pl.BlockSpec((R, _D), lambda i: (i, 0)),
          pl.BlockSpec((R, 1), lambda i: (i, 0)),
          pl.BlockSpec((R, 1), lambda i: (i, 0)),
          pl.BlockSpec((_D, _D), lambda i: (0, 0)),
          pl.BlockSpec((_D, _D), lambda i: (0, 0)),
          pl.BlockSpec((_D, _D), lambda i: (0, 0)),
          pl.BlockSpec((1, _D), lambda i: (0, 0)),
          pl.BlockSpec((R, _D), lambda i: (i, 0)),
      ],
      out_specs=[
          pl.BlockSpec((R, _D), lambda i: (i, 0)),
          pl.BlockSpec((8, _D), lambda i: (0, 0)),
      ],
      out_shape=[
          jax.ShapeDtypeStruct((n, _D), jnp.float32),
          jax.ShapeDtypeStruct((8, _D), jnp.float32),
      ],
      scratch_shapes=[pltpu.VMEM((8, _D), jnp.float32)],
      name=name,
  )


def _bn_relu_call(n, name):
  R = 1000
  grid = n // R

  def body(p, st, g, be, o_ref):
    m = st[0:1, :] / float(n)
    var = st[1:2, :] / float(n) - m * m
    scale = g[...] / jnp.sqrt(var + 1e-5)
    v = (p[...] - m) * scale + be[...]
    o_ref[...] = jnp.where(v >= 0, v, v * 0.01)

  return pl.pallas_call(
      body,
      grid=(grid,),
      in_specs=[
          pl.BlockSpec((R, _D), lambda i: (i, 0)),
          pl.BlockSpec((8, _D), lambda i: (0, 0)),
          pl.BlockSpec((1, _D), lambda i: (0, 0)),
          pl.BlockSpec((1, _D), lambda i: (0, 0)),
      ],
      out_specs=pl.BlockSpec((R, _D), lambda i: (i, 0)),
      out_shape=jax.ShapeDtypeStruct((n, _D), jnp.float32),
      name=name,
  )


def kernel(x_host, x_flow, edge_sends, edge_precedes, edge_rev_sends,
           edge_reaches,
           Wl_0_sends, bl_0_sends, Wr_0_sends,
           Wl_0_precedes, bl_0_precedes, Wr_0_precedes,
           Wl_0_rev_sends, bl_0_rev_sends, Wr_0_rev_sends,
           Wl_0_reaches, bl_0_reaches, Wr_0_reaches,
           g_0, be_0,
           Wl_1_sends, bl_1_sends, Wr_1_sends,
           Wl_1_precedes, bl_1_precedes, Wr_1_precedes,
           Wl_1_rev_sends, bl_1_rev_sends, Wr_1_rev_sends,
           Wl_1_reaches, bl_1_reaches, Wr_1_reaches,
           g_1, be_1):
  def _pack(e):
    return (e[0] | (e[1] << 16)).reshape(_NSUB, _SCAN_ROWS, _LANES)

  e4 = {
      "sends": _pack(edge_sends),
      "precedes": _pack(edge_precedes),
      "rev_sends": _pack(edge_rev_sends),
      "reaches": _pack(edge_reaches),
  }
  W = {
      0: dict(sends=(Wl_0_sends, bl_0_sends, Wr_0_sends),
              precedes=(Wl_0_precedes, bl_0_precedes, Wr_0_precedes),
              rev_sends=(Wl_0_rev_sends, bl_0_rev_sends, Wr_0_rev_sends),
              reaches=(Wl_0_reaches, bl_0_reaches, Wr_0_reaches)),
      1: dict(sends=(Wl_1_sends, bl_1_sends, Wr_1_sends),
              precedes=(Wl_1_precedes, bl_1_precedes, Wr_1_precedes),
              rev_sends=(Wl_1_rev_sends, bl_1_rev_sends, Wr_1_rev_sends),
              reaches=(Wl_1_reaches, bl_1_reaches, Wr_1_reaches)),
  }
  bn = {0: (g_0, be_0), 1: (g_1, be_1)}

  cS, cP, cR, cH = _cntk("cnt_all")(e4["sends"], e4["precedes"],
                                    e4["rev_sends"], e4["reaches"])
  cnt = {"sends": cS[:, 0:1], "precedes": cP[:, 0:1],
         "rev_sends": cR[:, 0:1], "reaches": cH[:, 0:1]}

  x = {"host": x_host, "flow": x_flow}
  for layer in (0, 1):
    aS, aP, aR, aH = _layerk("seg_layer")(
        x["host"], x["flow"], e4["sends"], e4["precedes"],
        e4["rev_sends"], e4["reaches"])
    agg = {"sends": aS, "precedes": aP, "rev_sends": aR, "reaches": aH}

    g, be = bn[layer]
    nxt = {}
    for t, (ra, rb), n in (("flow", ("sends", "precedes"), _N_FLOW),
                           ("host", ("rev_sends", "reaches"), _N_HOST)):
      WlA, blA, WrA = W[layer][ra]
      WlB, blB, WrB = W[layer][rb]
      wrc = 0.5 * (WrA + WrB)
      bc = (0.5 * (blA + blB)).reshape(1, _D)
      comb = _combine_stats_call(n, f"combine_{t}_{layer}")
      p, st = comb(agg[ra], agg[rb], cnt[ra], cnt[rb],
                   0.5 * WlA, 0.5 * WlB, wrc, bc, x[t])
      bnk = _bn_relu_call(n, f"bn_{t}_{layer}")
      nxt[t] = bnk(p, st, g.reshape(1, _D), be.reshape(1, _D))
    x = nxt

  return (x["flow"], x["host"])


# speedup vs baseline: 1.0332x; 1.0332x over previous
import functools

import jax
import jax.numpy as jnp
from jax import lax
from jax.experimental import pallas as pl
from jax.experimental.pallas import tpu as pltpu
from jax.experimental.pallas import tpu_sc as plsc

_N_HOST = 10000
_N_FLOW = 50000
_D = 128
_E = 160000

_NCORE = 2
_NSUB = 16
_LANES = 16

_EP = _E // _NSUB
_SCAN_ROWS = _EP // _LANES
_BATCH = 128
_NB_MAX = _EP // _BATCH

_SC_PARAMS = dict(
    compiler_params=pltpu.CompilerParams(needs_layout_passes=False,
                                         use_tc_tiling_on_sc=False))


def _sc_mesh():
  return plsc.VectorSubcoreMesh(core_axis_name="c", subcore_axis_name="s",
                                num_cores=_NCORE, num_subcores=_NSUB)


def _zero_rowbuf(rowbuf):
  z16 = jnp.zeros((_LANES,), jnp.float32)

  def zb(i, _):
    for k in range(_D // _LANES):
      rowbuf[i, pl.ds(k * _LANES, _LANES)] = z16
    return 0
  lax.fori_loop(0, _BATCH, zb, 0)


def _compact_chunk(ev, dstbuf, srcbuf, lo, ch, dump):
  iota = jnp.arange(_LANES, dtype=jnp.int32)
  zi16 = jnp.zeros((_LANES,), jnp.int32)

  def scan_body(j, posv):
    p16 = ev[j]
    d16 = lax.shift_right_logical(p16, jnp.full((_LANES,), 16, jnp.int32))
    inm = (d16 >= lo) & (d16 < lo + ch)
    ex = plsc.cumsum(inm.astype(jnp.int32))
    tgt = posv + ex - 1
    row = jnp.right_shift(tgt, 7)
    col = jnp.bitwise_and(tgt, _BATCH - 1)
    plsc.store_scatter(dstbuf, [row, col], d16 - lo, mask=inm)
    if srcbuf is not None:
      plsc.store_scatter(srcbuf, [row, col],
                         jnp.bitwise_and(p16, 0xFFFF), mask=inm)
    return posv + plsc.all_reduce_population_count(inm)
  posv = lax.fori_loop(0, _SCAN_ROWS, scan_body, zi16)

  nbv = jnp.right_shift(posv + (_BATCH - 1), 7)
  lastrow = nbv - 1
  for k in range(_BATCH // _LANES):
    colk = k * _LANES + iota
    flatp = lastrow * _BATCH + colk
    m = flatp >= posv
    plsc.store_scatter(dstbuf, [lastrow, colk],
                       jnp.full((_LANES,), dump, jnp.int32), mask=m)
    if srcbuf is not None:
      plsc.store_scatter(srcbuf, [lastrow, colk], zi16, mask=m)
  return jnp.max(nbv)


_CH_FLOW = 6400
_CH_HOST = 5120
_NPAD_FLOW = 8 * _CH_FLOW
_NPAD_HOST = 2 * _CH_HOST
_CHC_FLOW = 25008
_CHC_HOST = 5008


def _agg_relation(x_hbm, e_hbm, agg_hbm, ch, n_dst, refs):
  (ev, srcbuf, dstbuf, bufs, gsems, agg_s) = refs
  cid = lax.axis_index("c")
  sid = lax.axis_index("s")
  nchunk = -(-n_dst // ch)
  assert nchunk % _NCORE == 0 and ch % _NSUB == 0
  dump = ch
  rps = ch // _NSUB
  assert rps % 8 == 0

  pltpu.sync_copy(e_hbm.at[sid], ev)

  for p in range(nchunk // _NCORE):
    chunk = cid + _NCORE * p
    lo = chunk * ch

    _zero_rowbuf(bufs[0])
    rem = rps % _BATCH
    for k in range(rps // _BATCH):
      pltpu.sync_copy(bufs[0], agg_s.at[pl.ds(sid * rps + k * _BATCH,
                                              _BATCH)])
    if rem:
      pltpu.sync_copy(
          bufs[0].at[pl.ds(0, rem)],
          agg_s.at[pl.ds(sid * rps + (rps // _BATCH) * _BATCH, rem)])
    plsc.subcore_barrier()

    nb = _compact_chunk(ev, dstbuf, srcbuf, lo, ch, dump)

    for q in range(3):
      @pl.when(q < nb)
      def _(q=q):
        pltpu.async_copy(x_hbm.at[srcbuf.at[q]], bufs[q], gsems[q])

    def bat(g, _):
      for q in range(3):
        b = 3 * g + q

        @pl.when(b < nb)
        def _(b=b, q=q):
          pltpu.make_async_copy(x_hbm.at[srcbuf.at[b]], bufs[q],
                                gsems[q]).wait()
          pltpu.sync_copy(bufs[q], agg_s.at[dstbuf.at[b]], add=True)

          @pl.when(b + 3 < nb)
          def _():
            pltpu.async_copy(x_hbm.at[srcbuf.at[b + 3]], bufs[q],
                             gsems[q])
      return 0
    lax.fori_loop(0, (_NB_MAX + 2) // 3, bat, 0)

    plsc.subcore_barrier()

    base = lo + sid * rps
    for k in range(rps // _BATCH):
      pltpu.sync_copy(agg_s.at[pl.ds(sid * rps + k * _BATCH, _BATCH)],
                      agg_hbm.at[pl.ds(base + k * _BATCH, _BATCH)])
    if rem:
      pltpu.sync_copy(
          agg_s.at[pl.ds(sid * rps + (rps // _BATCH) * _BATCH, rem)],
          agg_hbm.at[pl.ds(base + (rps // _BATCH) * _BATCH, rem)])
    plsc.subcore_barrier()


def _make_pair_kernel(ch: int, n_dst: int, name: str):
  npad = (-(-n_dst // ch)) * ch
  out_type = (
      jax.ShapeDtypeStruct((npad, _D), jnp.float32),
      jax.ShapeDtypeStruct((npad, _D), jnp.float32),
  )
  scratch = dict(
      ev=pltpu.VMEM((_SCAN_ROWS, _LANES), jnp.int32),
      srcbuf=pltpu.VMEM((_NB_MAX, _BATCH), jnp.int32),
      dstbuf=pltpu.VMEM((_NB_MAX, _BATCH), jnp.int32),
      rowbuf0=pltpu.VMEM((_BATCH, _D), jnp.float32),
      rowbuf1=pltpu.VMEM((_BATCH, _D), jnp.float32),
      rowbuf2=pltpu.VMEM((_BATCH, _D), jnp.float32),
      agg_s=pltpu.VMEM_SHARED((ch + 16, _D), jnp.float32),
      gsem0=pltpu.SemaphoreType.DMA,
      gsem1=pltpu.SemaphoreType.DMA,
      gsem2=pltpu.SemaphoreType.DMA,
  )

  def body(xA_hbm, xB_hbm, eA, eB, aA, aB, *, ev, srcbuf,
           dstbuf, rowbuf0, rowbuf1, rowbuf2, agg_s, gsem0, gsem1, gsem2):
    refs = (ev, srcbuf, dstbuf, (rowbuf0, rowbuf1, rowbuf2),
            (gsem0, gsem1, gsem2), agg_s)
    _agg_relation(xA_hbm, eA, aA, ch, n_dst, refs)
    _agg_relation(xB_hbm, eB, aB, ch, n_dst, refs)

  return pl.kernel(body, out_type=out_type, mesh=_sc_mesh(),
                   scratch_types=scratch, name=name, **_SC_PARAMS)


def _make_cnt_kernel(name: str):
  scratch = dict(
      ev=pltpu.VMEM((_SCAN_ROWS, _LANES), jnp.int32),
      dstbuf=pltpu.VMEM((_NB_MAX, _BATCH), jnp.int32),
      onesb=pltpu.VMEM((_BATCH, 16), jnp.float32),
      zc=pltpu.VMEM((_BATCH, 16), jnp.float32),
      cnt_s=pltpu.VMEM_SHARED((_CHC_FLOW + 16, 16), jnp.float32),
      sem=pltpu.SemaphoreType.DMA,
  )

  def body(eS, eP, eR, eH, cS, cP, cR, cH, *, ev, dstbuf, onesb, zc,
           cnt_s, sem):
    cid = lax.axis_index("c")
    sid = lax.axis_index("s")
    iota = jnp.arange(_LANES, dtype=jnp.int32)
    one0 = (iota == 0).astype(jnp.float32)
    z16 = jnp.zeros((_LANES,), jnp.float32)

    def ob(i, _):
      onesb[i, pl.ds(0, _LANES)] = one0
      zc[i, pl.ds(0, _LANES)] = z16
      return 0
    lax.fori_loop(0, _BATCH, ob, 0)

    for e_hbm, c_hbm, ch in ((eS, cS, _CHC_FLOW), (eP, cP, _CHC_FLOW),
                             (eR, cR, _CHC_HOST), (eH, cH, _CHC_HOST)):
      rps = ch // _NSUB
      dump = ch
      lo = cid * ch
      pltpu.sync_copy(e_hbm.at[sid], ev)

      for k in range(rps // _BATCH):
        pltpu.sync_copy(zc, cnt_s.at[pl.ds(sid * rps + k * _BATCH, _BATCH)])
      rem = rps % _BATCH
      if rem:
        pltpu.sync_copy(
            zc.at[pl.ds(0, rem)],
            cnt_s.at[pl.ds(sid * rps + (rps // _BATCH) * _BATCH, rem)])
      plsc.subcore_barrier()

      nb = _compact_chunk(ev, dstbuf, None, lo, ch, dump)

      def fire(b, _):
        @pl.when(b < nb)
        def _():
          pltpu.async_copy(onesb, cnt_s.at[dstbuf.at[b]], sem, add=True)
        return 0
      lax.fori_loop(0, _NB_MAX, fire, 0)

      def drain(b, _):
        @pl.when(b < nb)
        def _():
          pltpu.make_async_copy(onesb, cnt_s.at[dstbuf.at[b]], sem).wait()
        return 0
      lax.fori_loop(0, _NB_MAX, drain, 0)

      plsc.subcore_barrier()

      base = lo + sid * rps
      pltpu.sync_copy(cnt_s.at[pl.ds(sid * rps, rps)],
                      c_hbm.at[pl.ds(base, rps)])
      plsc.subcore_barrier()

  return pl.kernel(
      body,
      out_type=(jax.ShapeDtypeStruct((_NCORE * _CHC_FLOW, 16), jnp.float32),
                jax.ShapeDtypeStruct((_NCORE * _CHC_FLOW, 16), jnp.float32),
                jax.ShapeDtypeStruct((_NCORE * _CHC_HOST, 16), jnp.float32),
                jax.ShapeDtypeStruct((_NCORE * _CHC_HOST, 16), jnp.float32)),
      mesh=_sc_mesh(), scratch_types=scratch, name=name, **_SC_PARAMS)


@functools.cache
def _pairk(ch, n_dst, name):
  return _make_pair_kernel(ch, n_dst, name)


@functools.cache
def _cntk(name):
  return _make_cnt_kernel(name)


def _combine_stats_call(n, name):
  R = 1000
  grid = n // R

  def body(aggA, aggB, cA, cB, wA, wB, wr, bc, x, p_ref, st_ref, acc):
    i = pl.program_id(0)
    mA = aggA[...] / jnp.maximum(cA[...], 1.0)
    mB = aggB[...] / jnp.maximum(cB[...], 1.0)
    p = (jnp.dot(mA, wA[...], preferred_element_type=jnp.float32)
         + jnp.dot(mB, wB[...], preferred_element_type=jnp.float32)
         + jnp.dot(x[...], wr[...], preferred_element_type=jnp.float32)
         + bc[...])
    p_ref[...] = p
    s = jnp.sum(p, axis=0, keepdims=True)
    sq = jnp.sum(p * p, axis=0, keepdims=True)

    @pl.when(i == 0)
    def _():
      acc[...] = jnp.zeros_like(acc)

    acc[0:1, :] += s
    acc[1:2, :] += sq

    @pl.when(i == grid - 1)
    def _():
      st_ref[...] = acc[...]

  return pl.pallas_call(
      body,
      grid=(grid,),
      in_specs=[
          pl.BlockSpec((R, _D), lambda i: (i, 0)),
          pl.BlockSpec((R, _D), lambda i: (i, 0)),
          pl.BlockSpec((R, 1), lambda i: (i, 0)),
          pl.BlockSpec((R, 1), lambda i: (i, 0)),
          pl.BlockSpec((_D, _D), lambda i: (0, 0)),
          pl.BlockSpec((_D, _D), lambda i: (0, 0)),
          pl.BlockSpec((_D, _D), lambda i: (0, 0)),
          pl.BlockSpec((1, _D), lambda i: (0, 0)),
          pl.BlockSpec((R, _D), lambda i: (i, 0)),
      ],
      out_specs=[
          pl.BlockSpec((R, _D), lambda i: (i, 0)),
          pl.BlockSpec((8, _D), lambda i: (0, 0)),
      ],
      out_shape=[
          jax.ShapeDtypeStruct((n, _D), jnp.float32),
          jax.ShapeDtypeStruct((8, _D), jnp.float32),
      ],
      scratch_shapes=[pltpu.VMEM((8, _D), jnp.float32)],
      name=name,
  )


def _bn_relu_call(n, name):
  R = 1000
  grid = n // R

  def body(p, st, g, be, o_ref):
    m = st[0:1, :] / float(n)
    var = st[1:2, :] / float(n) - m * m
    scale = g[...] / jnp.sqrt(var + 1e-5)
    v = (p[...] - m) * scale + be[...]
    o_ref[...] = jnp.where(v >= 0, v, v * 0.01)

  return pl.pallas_call(
      body,
      grid=(grid,),
      in_specs=[
          pl.BlockSpec((R, _D), lambda i: (i, 0)),
          pl.BlockSpec((8, _D), lambda i: (0, 0)),
          pl.BlockSpec((1, _D), lambda i: (0, 0)),
          pl.BlockSpec((1, _D), lambda i: (0, 0)),
      ],
      out_specs=pl.BlockSpec((R, _D), lambda i: (i, 0)),
      out_shape=jax.ShapeDtypeStruct((n, _D), jnp.float32),
      name=name,
  )


def kernel(x_host, x_flow, edge_sends, edge_precedes, edge_rev_sends,
           edge_reaches,
           Wl_0_sends, bl_0_sends, Wr_0_sends,
           Wl_0_precedes, bl_0_precedes, Wr_0_precedes,
           Wl_0_rev_sends, bl_0_rev_sends, Wr_0_rev_sends,
           Wl_0_reaches, bl_0_reaches, Wr_0_reaches,
           g_0, be_0,
           Wl_1_sends, bl_1_sends, Wr_1_sends,
           Wl_1_precedes, bl_1_precedes, Wr_1_precedes,
           Wl_1_rev_sends, bl_1_rev_sends, Wr_1_rev_sends,
           Wl_1_reaches, bl_1_reaches, Wr_1_reaches,
           g_1, be_1):
  def _pack(e):
    return (e[0] | (e[1] << 16)).reshape(_NSUB, _SCAN_ROWS, _LANES)

  e4 = {
      "sends": _pack(edge_sends),
      "precedes": _pack(edge_precedes),
      "rev_sends": _pack(edge_rev_sends),
      "reaches": _pack(edge_reaches),
  }
  W = {
      0: dict(sends=(Wl_0_sends, bl_0_sends, Wr_0_sends),
              precedes=(Wl_0_precedes, bl_0_precedes, Wr_0_precedes),
              rev_sends=(Wl_0_rev_sends, bl_0_rev_sends, Wr_0_rev_sends),
              reaches=(Wl_0_reaches, bl_0_reaches, Wr_0_reaches)),
      1: dict(sends=(Wl_1_sends, bl_1_sends, Wr_1_sends),
              precedes=(Wl_1_precedes, bl_1_precedes, Wr_1_precedes),
              rev_sends=(Wl_1_rev_sends, bl_1_rev_sends, Wr_1_rev_sends),
              reaches=(Wl_1_reaches, bl_1_reaches, Wr_1_reaches)),
  }
  bn = {0: (g_0, be_0), 1: (g_1, be_1)}

  cS, cP, cR, cH = _cntk("cnt_all")(e4["sends"], e4["precedes"],
                                    e4["rev_sends"], e4["reaches"])
  cnt = {"sends": cS[:, 0:1], "precedes": cP[:, 0:1],
         "rev_sends": cR[:, 0:1], "reaches": cH[:, 0:1]}

  x = {"host": x_host, "flow": x_flow}
  for layer in (0, 1):
    aS, aP = _pairk(_CH_FLOW, _N_FLOW, "seg_flow")(
        x["host"], x["flow"], e4["sends"], e4["precedes"])
    aR, aH = _pairk(_CH_HOST, _N_HOST, "seg_host")(
        x["flow"], x["flow"], e4["rev_sends"], e4["reaches"])
    agg = {"sends": aS, "precedes": aP, "rev_sends": aR, "reaches": aH}

    g, be = bn[layer]
    nxt = {}
    for t, (ra, rb), n in (("flow", ("sends", "precedes"), _N_FLOW),
                           ("host", ("rev_sends", "reaches"), _N_HOST)):
      WlA, blA, WrA = W[layer][ra]
      WlB, blB, WrB = W[layer][rb]
      wrc = 0.5 * (WrA + WrB)
      bc = (0.5 * (blA + blB)).reshape(1, _D)
      comb = _combine_stats_call(n, f"combine_{t}_{layer}")
      p, st = comb(agg[ra], agg[rb], cnt[ra], cnt[rb],
                   0.5 * WlA, 0.5 * WlB, wrc, bc, x[t])
      bnk = _bn_relu_call(n, f"bn_{t}_{layer}")
      nxt[t] = bnk(p, st, g.reshape(1, _D), be.reshape(1, _D))
    x = nxt

  return (x["flow"], x["host"])


# speedup vs baseline: 1.6874x; 1.6332x over previous
import functools

import jax
import jax.numpy as jnp
from jax import lax
from jax.experimental import pallas as pl
from jax.experimental.pallas import tpu as pltpu
from jax.experimental.pallas import tpu_sc as plsc

_N_HOST = 10000
_N_FLOW = 50000
_D = 128
_E = 160000

_NCORE = 2
_NSUB = 16
_LANES = 16

_EP = _E // _NSUB
_SCAN_ROWS = _EP // _LANES
_BATCH = 128
_NB_MAX = _EP // _BATCH

_SC_PARAMS = dict(
    compiler_params=pltpu.CompilerParams(needs_layout_passes=False,
                                         use_tc_tiling_on_sc=False))


def _sc_mesh():
  return plsc.VectorSubcoreMesh(core_axis_name="c", subcore_axis_name="s",
                                num_cores=_NCORE, num_subcores=_NSUB)


def _zero_rowbuf(rowbuf):
  z32 = jnp.zeros((2 * _LANES,), jnp.bfloat16)

  def zb(i, _):
    for k in range(_D // (2 * _LANES)):
      rowbuf[i, pl.ds(k * 2 * _LANES, 2 * _LANES)] = z32
    return 0
  lax.fori_loop(0, _BATCH, zb, 0)


def _compact_chunk(ev, dstbuf, srcbuf, lo, ch, dump):
  iota = jnp.arange(_LANES, dtype=jnp.int32)
  zi16 = jnp.zeros((_LANES,), jnp.int32)

  def scan_body(j, posv):
    p16 = ev[j]
    d16 = lax.shift_right_logical(p16, jnp.full((_LANES,), 16, jnp.int32))
    inm = (d16 >= lo) & (d16 < lo + ch)
    ex = plsc.cumsum(inm.astype(jnp.int32))
    tgt = posv + ex - 1
    row = jnp.right_shift(tgt, 7)
    col = jnp.bitwise_and(tgt, _BATCH - 1)
    plsc.store_scatter(dstbuf, [row, col], d16 - lo, mask=inm)
    if srcbuf is not None:
      plsc.store_scatter(srcbuf, [row, col],
                         jnp.bitwise_and(p16, 0xFFFF), mask=inm)
    return posv + plsc.all_reduce_population_count(inm)
  posv = lax.fori_loop(0, _SCAN_ROWS, scan_body, zi16)

  nbv = jnp.right_shift(posv + (_BATCH - 1), 7)
  lastrow = nbv - 1
  for k in range(_BATCH // _LANES):
    colk = k * _LANES + iota
    flatp = lastrow * _BATCH + colk
    m = flatp >= posv
    plsc.store_scatter(dstbuf, [lastrow, colk],
                       jnp.full((_LANES,), dump, jnp.int32), mask=m)
    if srcbuf is not None:
      plsc.store_scatter(srcbuf, [lastrow, colk], zi16, mask=m)
  return jnp.max(nbv)


_CH_FLOW = 12800
_CH_HOST = 5120
_NPAD_FLOW = 4 * _CH_FLOW
_NPAD_HOST = 2 * _CH_HOST
_CHC_FLOW = 25008
_CHC_HOST = 5008


def _agg_relation(x_hbm, e_hbm, agg_hbm, ch, n_dst, refs):
  (ev, srcbuf, dstbuf, bufs, gsems, agg_s) = refs
  cid = lax.axis_index("c")
  sid = lax.axis_index("s")
  nchunk = -(-n_dst // ch)
  assert nchunk % _NCORE == 0 and ch % _NSUB == 0
  dump = ch
  rps = ch // _NSUB
  assert rps % 8 == 0

  pltpu.sync_copy(e_hbm.at[sid], ev)

  for p in range(nchunk // _NCORE):
    chunk = cid + _NCORE * p
    lo = chunk * ch

    _zero_rowbuf(bufs[0])
    rem = rps % _BATCH
    for k in range(rps // _BATCH):
      pltpu.sync_copy(bufs[0], agg_s.at[pl.ds(sid * rps + k * _BATCH,
                                              _BATCH)])
    if rem:
      pltpu.sync_copy(
          bufs[0].at[pl.ds(0, rem)],
          agg_s.at[pl.ds(sid * rps + (rps // _BATCH) * _BATCH, rem)])
    plsc.subcore_barrier()

    nb = _compact_chunk(ev, dstbuf, srcbuf, lo, ch, dump)

    for q in range(3):
      @pl.when(q < nb)
      def _(q=q):
        pltpu.async_copy(x_hbm.at[srcbuf.at[q]], bufs[q], gsems[q])

    def bat(g, _):
      for q in range(3):
        b = 3 * g + q

        @pl.when(b < nb)
        def _(b=b, q=q):
          pltpu.make_async_copy(x_hbm.at[srcbuf.at[b]], bufs[q],
                                gsems[q]).wait()
          pltpu.sync_copy(bufs[q], agg_s.at[dstbuf.at[b]], add=True)

          @pl.when(b + 3 < nb)
          def _():
            pltpu.async_copy(x_hbm.at[srcbuf.at[b + 3]], bufs[q],
                             gsems[q])
      return 0
    lax.fori_loop(0, (_NB_MAX + 2) // 3, bat, 0)

    plsc.subcore_barrier()

    base = lo + sid * rps
    for k in range(rps // _BATCH):
      pltpu.sync_copy(agg_s.at[pl.ds(sid * rps + k * _BATCH, _BATCH)],
                      agg_hbm.at[pl.ds(base + k * _BATCH, _BATCH)])
    if rem:
      pltpu.sync_copy(
          agg_s.at[pl.ds(sid * rps + (rps // _BATCH) * _BATCH, rem)],
          agg_hbm.at[pl.ds(base + (rps // _BATCH) * _BATCH, rem)])
    plsc.subcore_barrier()


def _make_pair_kernel(ch: int, n_dst: int, name: str):
  npad = (-(-n_dst // ch)) * ch
  out_type = (
      jax.ShapeDtypeStruct((npad, _D), jnp.bfloat16),
      jax.ShapeDtypeStruct((npad, _D), jnp.bfloat16),
  )
  scratch = dict(
      ev=pltpu.VMEM((_SCAN_ROWS, _LANES), jnp.int32),
      srcbuf=pltpu.VMEM((_NB_MAX, _BATCH), jnp.int32),
      dstbuf=pltpu.VMEM((_NB_MAX, _BATCH), jnp.int32),
      rowbuf0=pltpu.VMEM((_BATCH, _D), jnp.bfloat16),
      rowbuf1=pltpu.VMEM((_BATCH, _D), jnp.bfloat16),
      rowbuf2=pltpu.VMEM((_BATCH, _D), jnp.bfloat16),
      agg_s=pltpu.VMEM_SHARED((ch + 16, _D), jnp.bfloat16),
      gsem0=pltpu.SemaphoreType.DMA,
      gsem1=pltpu.SemaphoreType.DMA,
      gsem2=pltpu.SemaphoreType.DMA,
  )

  def body(xA_hbm, xB_hbm, eA, eB, aA, aB, *, ev, srcbuf,
           dstbuf, rowbuf0, rowbuf1, rowbuf2, agg_s, gsem0, gsem1, gsem2):
    refs = (ev, srcbuf, dstbuf, (rowbuf0, rowbuf1, rowbuf2),
            (gsem0, gsem1, gsem2), agg_s)
    _agg_relation(xA_hbm, eA, aA, ch, n_dst, refs)
    _agg_relation(xB_hbm, eB, aB, ch, n_dst, refs)

  return pl.kernel(body, out_type=out_type, mesh=_sc_mesh(),
                   scratch_types=scratch, name=name, **_SC_PARAMS)


def _make_cnt_kernel(name: str):
  scratch = dict(
      ev=pltpu.VMEM((_SCAN_ROWS, _LANES), jnp.int32),
      dstbuf=pltpu.VMEM((_NB_MAX, _BATCH), jnp.int32),
      onesb=pltpu.VMEM((_BATCH, 16), jnp.float32),
      zc=pltpu.VMEM((_BATCH, 16), jnp.float32),
      cnt_s=pltpu.VMEM_SHARED((_CHC_FLOW + 16, 16), jnp.float32),
      sem=pltpu.SemaphoreType.DMA,
  )

  def body(eS, eP, eR, eH, cS, cP, cR, cH, *, ev, dstbuf, onesb, zc,
           cnt_s, sem):
    cid = lax.axis_index("c")
    sid = lax.axis_index("s")
    iota = jnp.arange(_LANES, dtype=jnp.int32)
    one0 = (iota == 0).astype(jnp.float32)
    z16 = jnp.zeros((_LANES,), jnp.float32)

    def ob(i, _):
      onesb[i, pl.ds(0, _LANES)] = one0
      zc[i, pl.ds(0, _LANES)] = z16
      return 0
    lax.fori_loop(0, _BATCH, ob, 0)

    for e_hbm, c_hbm, ch in ((eS, cS, _CHC_FLOW), (eP, cP, _CHC_FLOW),
                             (eR, cR, _CHC_HOST), (eH, cH, _CHC_HOST)):
      rps = ch // _NSUB
      dump = ch
      lo = cid * ch
      pltpu.sync_copy(e_hbm.at[sid], ev)

      for k in range(rps // _BATCH):
        pltpu.sync_copy(zc, cnt_s.at[pl.ds(sid * rps + k * _BATCH, _BATCH)])
      rem = rps % _BATCH
      if rem:
        pltpu.sync_copy(
            zc.at[pl.ds(0, rem)],
            cnt_s.at[pl.ds(sid * rps + (rps // _BATCH) * _BATCH, rem)])
      plsc.subcore_barrier()

      nb = _compact_chunk(ev, dstbuf, None, lo, ch, dump)

      def fire(b, _):
        @pl.when(b < nb)
        def _():
          pltpu.async_copy(onesb, cnt_s.at[dstbuf.at[b]], sem, add=True)
        return 0
      lax.fori_loop(0, _NB_MAX, fire, 0)

      def drain(b, _):
        @pl.when(b < nb)
        def _():
          pltpu.make_async_copy(onesb, cnt_s.at[dstbuf.at[b]], sem).wait()
        return 0
      lax.fori_loop(0, _NB_MAX, drain, 0)

      plsc.subcore_barrier()

      base = lo + sid * rps
      pltpu.sync_copy(cnt_s.at[pl.ds(sid * rps, rps)],
                      c_hbm.at[pl.ds(base, rps)])
      plsc.subcore_barrier()

  return pl.kernel(
      body,
      out_type=(jax.ShapeDtypeStruct((_NCORE * _CHC_FLOW, 16), jnp.float32),
                jax.ShapeDtypeStruct((_NCORE * _CHC_FLOW, 16), jnp.float32),
                jax.ShapeDtypeStruct((_NCORE * _CHC_HOST, 16), jnp.float32),
                jax.ShapeDtypeStruct((_NCORE * _CHC_HOST, 16), jnp.float32)),
      mesh=_sc_mesh(), scratch_types=scratch, name=name, **_SC_PARAMS)


@functools.cache
def _pairk(ch, n_dst, name):
  return _make_pair_kernel(ch, n_dst, name)


@functools.cache
def _cntk(name):
  return _make_cnt_kernel(name)


def _combine_stats_call(n, name):
  R = 1000
  grid = n // R

  def body(aggA, aggB, cA, cB, wA, wB, wr, bc, x, p_ref, st_ref, acc):
    i = pl.program_id(0)
    mA = aggA[...].astype(jnp.float32) / jnp.maximum(cA[...], 1.0)
    mB = aggB[...].astype(jnp.float32) / jnp.maximum(cB[...], 1.0)
    p = (jnp.dot(mA, wA[...], preferred_element_type=jnp.float32)
         + jnp.dot(mB, wB[...], preferred_element_type=jnp.float32)
         + jnp.dot(x[...], wr[...], preferred_element_type=jnp.float32)
         + bc[...])
    p_ref[...] = p
    s = jnp.sum(p, axis=0, keepdims=True)
    sq = jnp.sum(p * p, axis=0, keepdims=True)

    @pl.when(i == 0)
    def _():
      acc[...] = jnp.zeros_like(acc)

    acc[0:1, :] += s
    acc[1:2, :] += sq

    @pl.when(i == grid - 1)
    def _():
      st_ref[...] = acc[...]

  return pl.pallas_call(
      body,
      grid=(grid,),
      in_specs=[
          pl.BlockSpec((R, _D), lambda i: (i, 0)),
          pl.BlockSpec((R, _D), lambda i: (i, 0)),
          pl.BlockSpec((R, 1), lambda i: (i, 0)),
          pl.BlockSpec((R, 1), lambda i: (i, 0)),
          pl.BlockSpec((_D, _D), lambda i: (0, 0)),
          pl.BlockSpec((_D, _D), lambda i: (0, 0)),
          pl.BlockSpec((_D, _D), lambda i: (0, 0)),
          pl.BlockSpec((1, _D), lambda i: (0, 0)),
          pl.BlockSpec((R, _D), lambda i: (i, 0)),
      ],
      out_specs=[
          pl.BlockSpec((R, _D), lambda i: (i, 0)),
          pl.BlockSpec((8, _D), lambda i: (0, 0)),
      ],
      out_shape=[
          jax.ShapeDtypeStruct((n, _D), jnp.float32),
          jax.ShapeDtypeStruct((8, _D), jnp.float32),
      ],
      scratch_shapes=[pltpu.VMEM((8, _D), jnp.float32)],
      name=name,
  )


def _bn_relu_call(n, name):
  R = 1000
  grid = n // R

  def body(p, st, g, be, o_ref, o16_ref):
    m = st[0:1, :] / float(n)
    var = st[1:2, :] / float(n) - m * m
    scale = g[...] / jnp.sqrt(var + 1e-5)
    v = (p[...] - m) * scale + be[...]
    v = jnp.where(v >= 0, v, v * 0.01)
    o_ref[...] = v
    o16_ref[...] = v.astype(jnp.bfloat16)

  return pl.pallas_call(
      body,
      grid=(grid,),
      in_specs=[
          pl.BlockSpec((R, _D), lambda i: (i, 0)),
          pl.BlockSpec((8, _D), lambda i: (0, 0)),
          pl.BlockSpec((1, _D), lambda i: (0, 0)),
          pl.BlockSpec((1, _D), lambda i: (0, 0)),
      ],
      out_specs=[
          pl.BlockSpec((R, _D), lambda i: (i, 0)),
          pl.BlockSpec((R, _D), lambda i: (i, 0)),
      ],
      out_shape=[
          jax.ShapeDtypeStruct((n, _D), jnp.float32),
          jax.ShapeDtypeStruct((n, _D), jnp.bfloat16),
      ],
      name=name,
  )


def kernel(x_host, x_flow, edge_sends, edge_precedes, edge_rev_sends,
           edge_reaches,
           Wl_0_sends, bl_0_sends, Wr_0_sends,
           Wl_0_precedes, bl_0_precedes, Wr_0_precedes,
           Wl_0_rev_sends, bl_0_rev_sends, Wr_0_rev_sends,
           Wl_0_reaches, bl_0_reaches, Wr_0_reaches,
           g_0, be_0,
           Wl_1_sends, bl_1_sends, Wr_1_sends,
           Wl_1_precedes, bl_1_precedes, Wr_1_precedes,
           Wl_1_rev_sends, bl_1_rev_sends, Wr_1_rev_sends,
           Wl_1_reaches, bl_1_reaches, Wr_1_reaches,
           g_1, be_1):
  def _pack(e):
    return (e[0] | (e[1] << 16)).reshape(_NSUB, _SCAN_ROWS, _LANES)

  e4 = {
      "sends": _pack(edge_sends),
      "precedes": _pack(edge_precedes),
      "rev_sends": _pack(edge_rev_sends),
      "reaches": _pack(edge_reaches),
  }
  W = {
      0: dict(sends=(Wl_0_sends, bl_0_sends, Wr_0_sends),
              precedes=(Wl_0_precedes, bl_0_precedes, Wr_0_precedes),
              rev_sends=(Wl_0_rev_sends, bl_0_rev_sends, Wr_0_rev_sends),
              reaches=(Wl_0_reaches, bl_0_reaches, Wr_0_reaches)),
      1: dict(sends=(Wl_1_sends, bl_1_sends, Wr_1_sends),
              precedes=(Wl_1_precedes, bl_1_precedes, Wr_1_precedes),
              rev_sends=(Wl_1_rev_sends, bl_1_rev_sends, Wr_1_rev_sends),
              reaches=(Wl_1_reaches, bl_1_reaches, Wr_1_reaches)),
  }
  bn = {0: (g_0, be_0), 1: (g_1, be_1)}

  cS, cP, cR, cH = _cntk("cnt_all")(e4["sends"], e4["precedes"],
                                    e4["rev_sends"], e4["reaches"])
  cnt = {"sends": cS[:, 0:1], "precedes": cP[:, 0:1],
         "rev_sends": cR[:, 0:1], "reaches": cH[:, 0:1]}

  x = {"host": x_host, "flow": x_flow}
  x16 = {"host": x_host.astype(jnp.bfloat16),
         "flow": x_flow.astype(jnp.bfloat16)}
  for layer in (0, 1):
    aS, aP = _pairk(_CH_FLOW, _N_FLOW, "seg_flow")(
        x16["host"], x16["flow"], e4["sends"], e4["precedes"])
    aR, aH = _pairk(_CH_HOST, _N_HOST, "seg_host")(
        x16["flow"], x16["flow"], e4["rev_sends"], e4["reaches"])
    agg = {"sends": aS, "precedes": aP, "rev_sends": aR, "reaches": aH}

    g, be = bn[layer]
    nxt = {}
    for t, (ra, rb), n in (("flow", ("sends", "precedes"), _N_FLOW),
                           ("host", ("rev_sends", "reaches"), _N_HOST)):
      WlA, blA, WrA = W[layer][ra]
      WlB, blB, WrB = W[layer][rb]
      wrc = 0.5 * (WrA + WrB)
      bc = (0.5 * (blA + blB)).reshape(1, _D)
      comb = _combine_stats_call(n, f"combine_{t}_{layer}")
      p, st = comb(agg[ra], agg[rb], cnt[ra], cnt[rb],
                   0.5 * WlA, 0.5 * WlB, wrc, bc, x[t])
      bnk = _bn_relu_call(n, f"bn_{t}_{layer}")
      nxt[t] = bnk(p, st, g.reshape(1, _D), be.reshape(1, _D))
    x = {t: v[0] for t, v in nxt.items()}
    x16 = {t: v[1] for t, v in nxt.items()}

  return (x["flow"], x["host"])


# speedup vs baseline: 1.6879x; 1.0003x over previous
import functools

import jax
import jax.numpy as jnp
from jax import lax
from jax.experimental import pallas as pl
from jax.experimental.pallas import tpu as pltpu
from jax.experimental.pallas import tpu_sc as plsc

_N_HOST = 10000
_N_FLOW = 50000
_D = 128
_E = 160000

_NCORE = 2
_NSUB = 16
_LANES = 16

_EP = _E // _NSUB
_SCAN_ROWS = _EP // _LANES
_BATCH = 128
_NB_MAX = _EP // _BATCH

_SC_PARAMS = dict(
    compiler_params=pltpu.CompilerParams(needs_layout_passes=False,
                                         use_tc_tiling_on_sc=False))


def _sc_mesh():
  return plsc.VectorSubcoreMesh(core_axis_name="c", subcore_axis_name="s",
                                num_cores=_NCORE, num_subcores=_NSUB)


def _zero_rowbuf(rowbuf):
  z32 = jnp.zeros((2 * _LANES,), jnp.bfloat16)

  def zb(i, _):
    for k in range(_D // (2 * _LANES)):
      rowbuf[i, pl.ds(k * 2 * _LANES, 2 * _LANES)] = z32
    return 0
  lax.fori_loop(0, _BATCH, zb, 0)


def _compact_chunk(ev, dstbuf, srcbuf, lo, ch, dump):
  iota = jnp.arange(_LANES, dtype=jnp.int32)
  zi16 = jnp.zeros((_LANES,), jnp.int32)

  def scan_body(j, posv):
    p16 = ev[j]
    d16 = lax.shift_right_logical(p16, jnp.full((_LANES,), 16, jnp.int32))
    inm = (d16 >= lo) & (d16 < lo + ch)
    ex = plsc.cumsum(inm.astype(jnp.int32))
    tgt = posv + ex - 1
    row = jnp.right_shift(tgt, 7)
    col = jnp.bitwise_and(tgt, _BATCH - 1)
    plsc.store_scatter(dstbuf, [row, col], d16 - lo, mask=inm)
    if srcbuf is not None:
      plsc.store_scatter(srcbuf, [row, col],
                         jnp.bitwise_and(p16, 0xFFFF), mask=inm)
    return posv + plsc.all_reduce_population_count(inm)
  posv = lax.fori_loop(0, _SCAN_ROWS, scan_body, zi16)

  nbv = jnp.right_shift(posv + (_BATCH - 1), 7)
  lastrow = nbv - 1
  for k in range(_BATCH // _LANES):
    colk = k * _LANES + iota
    flatp = lastrow * _BATCH + colk
    m = flatp >= posv
    plsc.store_scatter(dstbuf, [lastrow, colk],
                       jnp.full((_LANES,), dump, jnp.int32), mask=m)
    if srcbuf is not None:
      plsc.store_scatter(srcbuf, [lastrow, colk], zi16, mask=m)
  return jnp.max(nbv)


_CH_FLOW = 12800
_CH_HOST = 5120
_NPAD_FLOW = 4 * _CH_FLOW
_NPAD_HOST = 2 * _CH_HOST
_CHC_FLOW = 25008
_CHC_HOST = 5008


def _agg_relation(x_hbm, e_hbm, agg_hbm, ch, n_dst, refs):
  (ev, srcbuf, dstbuf, bufs, gsems, agg_s) = refs
  cid = lax.axis_index("c")
  sid = lax.axis_index("s")
  nchunk = -(-n_dst // ch)
  assert nchunk % _NCORE == 0 and ch % _NSUB == 0
  dump = ch
  rps = ch // _NSUB
  assert rps % 8 == 0

  pltpu.sync_copy(e_hbm.at[sid], ev)

  for p in range(nchunk // _NCORE):
    chunk = cid + _NCORE * p
    lo = chunk * ch

    _zero_rowbuf(bufs[0])
    rem = rps % _BATCH
    for k in range(rps // _BATCH):
      pltpu.sync_copy(bufs[0], agg_s.at[pl.ds(sid * rps + k * _BATCH,
                                              _BATCH)])
    if rem:
      pltpu.sync_copy(
          bufs[0].at[pl.ds(0, rem)],
          agg_s.at[pl.ds(sid * rps + (rps // _BATCH) * _BATCH, rem)])
    plsc.subcore_barrier()

    nb = _compact_chunk(ev, dstbuf, srcbuf, lo, ch, dump)

    for q in range(3):
      @pl.when(q < nb)
      def _(q=q):
        pltpu.async_copy(x_hbm.at[srcbuf.at[q]], bufs[q], gsems[q])

    def bat(g, _):
      for q in range(3):
        b = 3 * g + q

        @pl.when(b < nb)
        def _(b=b, q=q):
          pltpu.make_async_copy(x_hbm.at[srcbuf.at[b]], bufs[q],
                                gsems[q]).wait()
          pltpu.sync_copy(bufs[q], agg_s.at[dstbuf.at[b]], add=True)

          @pl.when(b + 3 < nb)
          def _():
            pltpu.async_copy(x_hbm.at[srcbuf.at[b + 3]], bufs[q],
                             gsems[q])
      return 0
    lax.fori_loop(0, (_NB_MAX + 2) // 3, bat, 0)

    plsc.subcore_barrier()

    base = lo + sid * rps
    for k in range(rps // _BATCH):
      pltpu.sync_copy(agg_s.at[pl.ds(sid * rps + k * _BATCH, _BATCH)],
                      agg_hbm.at[pl.ds(base + k * _BATCH, _BATCH)])
    if rem:
      pltpu.sync_copy(
          agg_s.at[pl.ds(sid * rps + (rps // _BATCH) * _BATCH, rem)],
          agg_hbm.at[pl.ds(base + (rps // _BATCH) * _BATCH, rem)])
    plsc.subcore_barrier()


def _make_pair_kernel(ch: int, n_dst: int, name: str):
  npad = (-(-n_dst // ch)) * ch
  out_type = (
      jax.ShapeDtypeStruct((npad, _D), jnp.bfloat16),
      jax.ShapeDtypeStruct((npad, _D), jnp.bfloat16),
  )
  scratch = dict(
      ev=pltpu.VMEM((_SCAN_ROWS, _LANES), jnp.int32),
      srcbuf=pltpu.VMEM((_NB_MAX, _BATCH), jnp.int32),
      dstbuf=pltpu.VMEM((_NB_MAX, _BATCH), jnp.int32),
      rowbuf0=pltpu.VMEM((_BATCH, _D), jnp.bfloat16),
      rowbuf1=pltpu.VMEM((_BATCH, _D), jnp.bfloat16),
      rowbuf2=pltpu.VMEM((_BATCH, _D), jnp.bfloat16),
      agg_s=pltpu.VMEM_SHARED((ch + 16, _D), jnp.bfloat16),
      gsem0=pltpu.SemaphoreType.DMA,
      gsem1=pltpu.SemaphoreType.DMA,
      gsem2=pltpu.SemaphoreType.DMA,
  )

  def body(xA_hbm, xB_hbm, eA, eB, aA, aB, *, ev, srcbuf,
           dstbuf, rowbuf0, rowbuf1, rowbuf2, agg_s, gsem0, gsem1, gsem2):
    refs = (ev, srcbuf, dstbuf, (rowbuf0, rowbuf1, rowbuf2),
            (gsem0, gsem1, gsem2), agg_s)
    _agg_relation(xA_hbm, eA, aA, ch, n_dst, refs)
    _agg_relation(xB_hbm, eB, aB, ch, n_dst, refs)

  return pl.kernel(body, out_type=out_type, mesh=_sc_mesh(),
                   scratch_types=scratch, name=name, **_SC_PARAMS)


def _make_cnt_kernel(name: str):
  scratch = dict(
      ev=pltpu.VMEM((_SCAN_ROWS, _LANES), jnp.int32),
      dstbuf=pltpu.VMEM((_NB_MAX, _BATCH), jnp.int32),
      onesb=pltpu.VMEM((_BATCH, 16), jnp.float32),
      zc=pltpu.VMEM((_BATCH, 16), jnp.float32),
      cnt_s=pltpu.VMEM_SHARED((_CHC_FLOW + 16, 16), jnp.float32),
      sem=pltpu.SemaphoreType.DMA,
  )

  def body(eS, eP, eR, eH, cS, cP, cR, cH, *, ev, dstbuf, onesb, zc,
           cnt_s, sem):
    cid = lax.axis_index("c")
    sid = lax.axis_index("s")
    iota = jnp.arange(_LANES, dtype=jnp.int32)
    one0 = (iota == 0).astype(jnp.float32)
    z16 = jnp.zeros((_LANES,), jnp.float32)

    def ob(i, _):
      onesb[i, pl.ds(0, _LANES)] = one0
      zc[i, pl.ds(0, _LANES)] = z16
      return 0
    lax.fori_loop(0, _BATCH, ob, 0)

    for e_hbm, c_hbm, ch in ((eS, cS, _CHC_FLOW), (eP, cP, _CHC_FLOW),
                             (eR, cR, _CHC_HOST), (eH, cH, _CHC_HOST)):
      rps = ch // _NSUB
      dump = ch
      lo = cid * ch
      pltpu.sync_copy(e_hbm.at[sid], ev)

      for k in range(rps // _BATCH):
        pltpu.sync_copy(zc, cnt_s.at[pl.ds(sid * rps + k * _BATCH, _BATCH)])
      rem = rps % _BATCH
      if rem:
        pltpu.sync_copy(
            zc.at[pl.ds(0, rem)],
            cnt_s.at[pl.ds(sid * rps + (rps // _BATCH) * _BATCH, rem)])
      plsc.subcore_barrier()

      nb = _compact_chunk(ev, dstbuf, None, lo, ch, dump)

      def fire(b, _):
        @pl.when(b < nb)
        def _():
          pltpu.async_copy(onesb, cnt_s.at[dstbuf.at[b]], sem, add=True)
        return 0
      lax.fori_loop(0, _NB_MAX, fire, 0)

      def drain(b, _):
        @pl.when(b < nb)
        def _():
          pltpu.make_async_copy(onesb, cnt_s.at[dstbuf.at[b]], sem).wait()
        return 0
      lax.fori_loop(0, _NB_MAX, drain, 0)

      plsc.subcore_barrier()

      base = lo + sid * rps
      pltpu.sync_copy(cnt_s.at[pl.ds(sid * rps, rps)],
                      c_hbm.at[pl.ds(base, rps)])
      plsc.subcore_barrier()

  return pl.kernel(
      body,
      out_type=(jax.ShapeDtypeStruct((_NCORE * _CHC_FLOW, 16), jnp.float32),
                jax.ShapeDtypeStruct((_NCORE * _CHC_FLOW, 16), jnp.float32),
                jax.ShapeDtypeStruct((_NCORE * _CHC_HOST, 16), jnp.float32),
                jax.ShapeDtypeStruct((_NCORE * _CHC_HOST, 16), jnp.float32)),
      mesh=_sc_mesh(), scratch_types=scratch, name=name, **_SC_PARAMS)


@functools.cache
def _pairk(ch, n_dst, name):
  return _make_pair_kernel(ch, n_dst, name)


@functools.cache
def _cntk(name):
  return _make_cnt_kernel(name)


def _combine_stats_call(n, name):
  R = 1000
  grid = n // R

  def body(aggA, aggB, cA, cB, wA, wB, wr, bc, x, p_ref, st_ref, acc):
    i = pl.program_id(0)
    mA = aggA[...].astype(jnp.float32) / jnp.maximum(cA[:, 0:1], 1.0)
    mB = aggB[...].astype(jnp.float32) / jnp.maximum(cB[:, 0:1], 1.0)
    p = (jnp.dot(mA, wA[...], preferred_element_type=jnp.float32)
         + jnp.dot(mB, wB[...], preferred_element_type=jnp.float32)
         + jnp.dot(x[...], wr[...], preferred_element_type=jnp.float32)
         + bc[...])
    p_ref[...] = p
    s = jnp.sum(p, axis=0, keepdims=True)
    sq = jnp.sum(p * p, axis=0, keepdims=True)

    @pl.when(i == 0)
    def _():
      acc[...] = jnp.zeros_like(acc)

    acc[0:1, :] += s
    acc[1:2, :] += sq

    @pl.when(i == grid - 1)
    def _():
      st_ref[...] = acc[...]

  return pl.pallas_call(
      body,
      grid=(grid,),
      in_specs=[
          pl.BlockSpec((R, _D), lambda i: (i, 0)),
          pl.BlockSpec((R, _D), lambda i: (i, 0)),
          pl.BlockSpec((R, 16), lambda i: (i, 0)),
          pl.BlockSpec((R, 16), lambda i: (i, 0)),
          pl.BlockSpec((_D, _D), lambda i: (0, 0)),
          pl.BlockSpec((_D, _D), lambda i: (0, 0)),
          pl.BlockSpec((_D, _D), lambda i: (0, 0)),
          pl.BlockSpec((1, _D), lambda i: (0, 0)),
          pl.BlockSpec((R, _D), lambda i: (i, 0)),
      ],
      out_specs=[
          pl.BlockSpec((R, _D), lambda i: (i, 0)),
          pl.BlockSpec((8, _D), lambda i: (0, 0)),
      ],
      out_shape=[
          jax.ShapeDtypeStruct((n, _D), jnp.float32),
          jax.ShapeDtypeStruct((8, _D), jnp.float32),
      ],
      scratch_shapes=[pltpu.VMEM((8, _D), jnp.float32)],
      name=name,
  )


def _bn_relu_call(n, name):
  R = 1000
  grid = n // R

  def body(p, st, g, be, o_ref, o16_ref):
    m = st[0:1, :] / float(n)
    var = st[1:2, :] / float(n) - m * m
    scale = g[...] / jnp.sqrt(var + 1e-5)
    v = (p[...] - m) * scale + be[...]
    v = jnp.where(v >= 0, v, v * 0.01)
    o_ref[...] = v
    o16_ref[...] = v.astype(jnp.bfloat16)

  return pl.pallas_call(
      body,
      grid=(grid,),
      in_specs=[
          pl.BlockSpec((R, _D), lambda i: (i, 0)),
          pl.BlockSpec((8, _D), lambda i: (0, 0)),
          pl.BlockSpec((1, _D), lambda i: (0, 0)),
          pl.BlockSpec((1, _D), lambda i: (0, 0)),
      ],
      out_specs=[
          pl.BlockSpec((R, _D), lambda i: (i, 0)),
          pl.BlockSpec((R, _D), lambda i: (i, 0)),
      ],
      out_shape=[
          jax.ShapeDtypeStruct((n, _D), jnp.float32),
          jax.ShapeDtypeStruct((n, _D), jnp.bfloat16),
      ],
      name=name,
  )


def kernel(x_host, x_flow, edge_sends, edge_precedes, edge_rev_sends,
           edge_reaches,
           Wl_0_sends, bl_0_sends, Wr_0_sends,
           Wl_0_precedes, bl_0_precedes, Wr_0_precedes,
           Wl_0_rev_sends, bl_0_rev_sends, Wr_0_rev_sends,
           Wl_0_reaches, bl_0_reaches, Wr_0_reaches,
           g_0, be_0,
           Wl_1_sends, bl_1_sends, Wr_1_sends,
           Wl_1_precedes, bl_1_precedes, Wr_1_precedes,
           Wl_1_rev_sends, bl_1_rev_sends, Wr_1_rev_sends,
           Wl_1_reaches, bl_1_reaches, Wr_1_reaches,
           g_1, be_1):
  def _pack(e):
    return (e[0] | (e[1] << 16)).reshape(_NSUB, _SCAN_ROWS, _LANES)

  e4 = {
      "sends": _pack(edge_sends),
      "precedes": _pack(edge_precedes),
      "rev_sends": _pack(edge_rev_sends),
      "reaches": _pack(edge_reaches),
  }
  W = {
      0: dict(sends=(Wl_0_sends, bl_0_sends, Wr_0_sends),
              precedes=(Wl_0_precedes, bl_0_precedes, Wr_0_precedes),
              rev_sends=(Wl_0_rev_sends, bl_0_rev_sends, Wr_0_rev_sends),
              reaches=(Wl_0_reaches, bl_0_reaches, Wr_0_reaches)),
      1: dict(sends=(Wl_1_sends, bl_1_sends, Wr_1_sends),
              precedes=(Wl_1_precedes, bl_1_precedes, Wr_1_precedes),
              rev_sends=(Wl_1_rev_sends, bl_1_rev_sends, Wr_1_rev_sends),
              reaches=(Wl_1_reaches, bl_1_reaches, Wr_1_reaches)),
  }
  bn = {0: (g_0, be_0), 1: (g_1, be_1)}

  cS, cP, cR, cH = _cntk("cnt_all")(e4["sends"], e4["precedes"],
                                    e4["rev_sends"], e4["reaches"])
  cnt = {"sends": cS, "precedes": cP, "rev_sends": cR, "reaches": cH}

  x = {"host": x_host, "flow": x_flow}
  x16 = {"host": x_host.astype(jnp.bfloat16),
         "flow": x_flow.astype(jnp.bfloat16)}
  for layer in (0, 1):
    aS, aP = _pairk(_CH_FLOW, _N_FLOW, "seg_flow")(
        x16["host"], x16["flow"], e4["sends"], e4["precedes"])
    aR, aH = _pairk(_CH_HOST, _N_HOST, "seg_host")(
        x16["flow"], x16["flow"], e4["rev_sends"], e4["reaches"])
    agg = {"sends": aS, "precedes": aP, "rev_sends": aR, "reaches": aH}

    g, be = bn[layer]
    nxt = {}
    for t, (ra, rb), n in (("flow", ("sends", "precedes"), _N_FLOW),
                           ("host", ("rev_sends", "reaches"), _N_HOST)):
      WlA, blA, WrA = W[layer][ra]
      WlB, blB, WrB = W[layer][rb]
      wrc = 0.5 * (WrA + WrB)
      bc = (0.5 * (blA + blB)).reshape(1, _D)
      comb = _combine_stats_call(n, f"combine_{t}_{layer}")
      p, st = comb(agg[ra], agg[rb], cnt[ra], cnt[rb],
                   0.5 * WlA, 0.5 * WlB, wrc, bc, x[t])
      bnk = _bn_relu_call(n, f"bn_{t}_{layer}")
      nxt[t] = bnk(p, st, g.reshape(1, _D), be.reshape(1, _D))
    x = {t: v[0] for t, v in nxt.items()}
    x16 = {t: v[1] for t, v in nxt.items()}

  return (x["flow"], x["host"])


# speedup vs baseline: 1.7528x; 1.0384x over previous
import functools

import jax
import jax.numpy as jnp
from jax import lax
from jax.experimental import pallas as pl
from jax.experimental.pallas import tpu as pltpu
from jax.experimental.pallas import tpu_sc as plsc

_N_HOST = 10000
_N_FLOW = 50000
_D = 128
_E = 160000

_NCORE = 2
_NSUB = 16
_LANES = 16

_EP = _E // _NSUB
_SCAN_ROWS = _EP // _LANES
_BATCH = 128
_NB_MAX = _EP // _BATCH

_SC_PARAMS = dict(
    compiler_params=pltpu.CompilerParams(needs_layout_passes=False,
                                         use_tc_tiling_on_sc=False))


def _sc_mesh():
  return plsc.VectorSubcoreMesh(core_axis_name="c", subcore_axis_name="s",
                                num_cores=_NCORE, num_subcores=_NSUB)


def _zero_rowbuf(rowbuf):
  z32 = jnp.zeros((2 * _LANES,), jnp.bfloat16)

  def zb(i, _):
    for k in range(_D // (2 * _LANES)):
      rowbuf[i, pl.ds(k * 2 * _LANES, 2 * _LANES)] = z32
    return 0
  lax.fori_loop(0, _BATCH, zb, 0)


def _compact_chunk(ev, dstbuf, srcbuf, lo, ch, dump):
  iota = jnp.arange(_LANES, dtype=jnp.int32)
  zi16 = jnp.zeros((_LANES,), jnp.int32)

  def scan_body(j, posv):
    p16 = ev[j]
    d16 = lax.shift_right_logical(p16, jnp.full((_LANES,), 16, jnp.int32))
    inm = (d16 >= lo) & (d16 < lo + ch)
    ex = plsc.cumsum(inm.astype(jnp.int32))
    tgt = posv + ex - 1
    row = jnp.right_shift(tgt, 7)
    col = jnp.bitwise_and(tgt, _BATCH - 1)
    plsc.store_scatter(dstbuf, [row, col], d16 - lo, mask=inm)
    if srcbuf is not None:
      plsc.store_scatter(srcbuf, [row, col],
                         jnp.bitwise_and(p16, 0xFFFF), mask=inm)
    return posv + plsc.all_reduce_population_count(inm)
  posv = lax.fori_loop(0, _SCAN_ROWS, scan_body, zi16)

  nbv = jnp.right_shift(posv + (_BATCH - 1), 7)
  lastrow = nbv - 1
  for k in range(_BATCH // _LANES):
    colk = k * _LANES + iota
    flatp = lastrow * _BATCH + colk
    m = flatp >= posv
    plsc.store_scatter(dstbuf, [lastrow, colk],
                       jnp.full((_LANES,), dump, jnp.int32), mask=m)
    if srcbuf is not None:
      plsc.store_scatter(srcbuf, [lastrow, colk], zi16, mask=m)
  return jnp.max(nbv)


_CH_FLOW = 12800
_CH_HOST = 5120
_NPAD_FLOW = 4 * _CH_FLOW
_NPAD_HOST = 2 * _CH_HOST
_CHC_FLOW = 25008
_CHC_HOST = 5008


def _agg_relation(x_hbm, e_hbm, agg_hbm, ch, n_dst, refs):
  (ev, srcbuf, dstbuf, bufs, gsems, agg_s) = refs
  cid = lax.axis_index("c")
  sid = lax.axis_index("s")
  nchunk = -(-n_dst // ch)
  assert nchunk % _NCORE == 0 and ch % _NSUB == 0
  dump = ch
  rps = ch // _NSUB
  assert rps % 8 == 0

  pltpu.sync_copy(e_hbm.at[sid], ev)

  for p in range(nchunk // _NCORE):
    chunk = cid + _NCORE * p
    lo = chunk * ch

    _zero_rowbuf(bufs[0])
    rem = rps % _BATCH
    for k in range(rps // _BATCH):
      pltpu.sync_copy(bufs[0], agg_s.at[pl.ds(sid * rps + k * _BATCH,
                                              _BATCH)])
    if rem:
      pltpu.sync_copy(
          bufs[0].at[pl.ds(0, rem)],
          agg_s.at[pl.ds(sid * rps + (rps // _BATCH) * _BATCH, rem)])
    plsc.subcore_barrier()

    nb = _compact_chunk(ev, dstbuf, srcbuf, lo, ch, dump)

    for q in range(3):
      @pl.when(q < nb)
      def _(q=q):
        pltpu.async_copy(x_hbm.at[srcbuf.at[q]], bufs[q], gsems[q])

    def bat(g, _):
      for q in range(3):
        b = 3 * g + q

        @pl.when(b < nb)
        def _(b=b, q=q):
          pltpu.make_async_copy(x_hbm.at[srcbuf.at[b]], bufs[q],
                                gsems[q]).wait()
          pltpu.sync_copy(bufs[q], agg_s.at[dstbuf.at[b]], add=True)

          @pl.when(b + 3 < nb)
          def _():
            pltpu.async_copy(x_hbm.at[srcbuf.at[b + 3]], bufs[q],
                             gsems[q])
      return 0
    lax.fori_loop(0, (_NB_MAX + 2) // 3, bat, 0)

    plsc.subcore_barrier()

    base = lo + sid * rps
    for k in range(rps // _BATCH):
      pltpu.sync_copy(agg_s.at[pl.ds(sid * rps + k * _BATCH, _BATCH)],
                      agg_hbm.at[pl.ds(base + k * _BATCH, _BATCH)])
    if rem:
      pltpu.sync_copy(
          agg_s.at[pl.ds(sid * rps + (rps // _BATCH) * _BATCH, rem)],
          agg_hbm.at[pl.ds(base + (rps // _BATCH) * _BATCH, rem)])
    plsc.subcore_barrier()


def _make_pair_kernel(ch: int, n_dst: int, name: str):
  npad = (-(-n_dst // ch)) * ch
  out_type = (
      jax.ShapeDtypeStruct((npad, _D), jnp.bfloat16),
      jax.ShapeDtypeStruct((npad, _D), jnp.bfloat16),
  )
  scratch = dict(
      ev=pltpu.VMEM((_SCAN_ROWS, _LANES), jnp.int32),
      srcbuf=pltpu.VMEM((_NB_MAX, _BATCH), jnp.int32),
      dstbuf=pltpu.VMEM((_NB_MAX, _BATCH), jnp.int32),
      rowbuf0=pltpu.VMEM((_BATCH, _D), jnp.bfloat16),
      rowbuf1=pltpu.VMEM((_BATCH, _D), jnp.bfloat16),
      rowbuf2=pltpu.VMEM((_BATCH, _D), jnp.bfloat16),
      agg_s=pltpu.VMEM_SHARED((ch + 16, _D), jnp.bfloat16),
      gsem0=pltpu.SemaphoreType.DMA,
      gsem1=pltpu.SemaphoreType.DMA,
      gsem2=pltpu.SemaphoreType.DMA,
  )

  def body(xA_hbm, xB_hbm, eA, eB, aA, aB, *, ev, srcbuf,
           dstbuf, rowbuf0, rowbuf1, rowbuf2, agg_s, gsem0, gsem1, gsem2):
    refs = (ev, srcbuf, dstbuf, (rowbuf0, rowbuf1, rowbuf2),
            (gsem0, gsem1, gsem2), agg_s)
    _agg_relation(xA_hbm, eA, aA, ch, n_dst, refs)
    _agg_relation(xB_hbm, eB, aB, ch, n_dst, refs)

  return pl.kernel(body, out_type=out_type, mesh=_sc_mesh(),
                   scratch_types=scratch, name=name, **_SC_PARAMS)


def _make_cnt_kernel(name: str):
  scratch = dict(
      ev=pltpu.VMEM((_SCAN_ROWS, _LANES), jnp.int32),
      dstbuf=pltpu.VMEM((_NB_MAX, _BATCH), jnp.int32),
      onesb=pltpu.VMEM((_BATCH, 16), jnp.float32),
      zc=pltpu.VMEM((_BATCH, 16), jnp.float32),
      cnt_s=pltpu.VMEM_SHARED((_CHC_FLOW + 16, 16), jnp.float32),
      sem=pltpu.SemaphoreType.DMA,
  )

  def body(eS, eP, eR, eH, cS, cP, cR, cH, *, ev, dstbuf, onesb, zc,
           cnt_s, sem):
    cid = lax.axis_index("c")
    sid = lax.axis_index("s")
    iota = jnp.arange(_LANES, dtype=jnp.int32)
    one0 = (iota == 0).astype(jnp.float32)
    z16 = jnp.zeros((_LANES,), jnp.float32)

    def ob(i, _):
      onesb[i, pl.ds(0, _LANES)] = one0
      zc[i, pl.ds(0, _LANES)] = z16
      return 0
    lax.fori_loop(0, _BATCH, ob, 0)

    for e_hbm, c_hbm, ch in ((eS, cS, _CHC_FLOW), (eP, cP, _CHC_FLOW),
                             (eR, cR, _CHC_HOST), (eH, cH, _CHC_HOST)):
      rps = ch // _NSUB
      dump = ch
      lo = cid * ch
      pltpu.sync_copy(e_hbm.at[sid], ev)

      for k in range(rps // _BATCH):
        pltpu.sync_copy(zc, cnt_s.at[pl.ds(sid * rps + k * _BATCH, _BATCH)])
      rem = rps % _BATCH
      if rem:
        pltpu.sync_copy(
            zc.at[pl.ds(0, rem)],
            cnt_s.at[pl.ds(sid * rps + (rps // _BATCH) * _BATCH, rem)])
      plsc.subcore_barrier()

      nb = _compact_chunk(ev, dstbuf, None, lo, ch, dump)

      def fire(b, _):
        @pl.when(b < nb)
        def _():
          pltpu.async_copy(onesb, cnt_s.at[dstbuf.at[b]], sem, add=True)
        return 0
      lax.fori_loop(0, _NB_MAX, fire, 0)

      def drain(b, _):
        @pl.when(b < nb)
        def _():
          pltpu.make_async_copy(onesb, cnt_s.at[dstbuf.at[b]], sem).wait()
        return 0
      lax.fori_loop(0, _NB_MAX, drain, 0)

      plsc.subcore_barrier()

      base = lo + sid * rps
      pltpu.sync_copy(cnt_s.at[pl.ds(sid * rps, rps)],
                      c_hbm.at[pl.ds(base, rps)])
      plsc.subcore_barrier()

  return pl.kernel(
      body,
      out_type=(jax.ShapeDtypeStruct((_NCORE * _CHC_FLOW, 16), jnp.float32),
                jax.ShapeDtypeStruct((_NCORE * _CHC_FLOW, 16), jnp.float32),
                jax.ShapeDtypeStruct((_NCORE * _CHC_HOST, 16), jnp.float32),
                jax.ShapeDtypeStruct((_NCORE * _CHC_HOST, 16), jnp.float32)),
      mesh=_sc_mesh(), scratch_types=scratch, name=name, **_SC_PARAMS)


@functools.cache
def _pairk(ch, n_dst, name):
  return _make_pair_kernel(ch, n_dst, name)


@functools.cache
def _cntk(name):
  return _make_cnt_kernel(name)


def _combine_stats_call(n, name):
  R = 2000
  grid = n // R

  def body(aggA, aggB, cA, cB, wA, wB, wr, bc, x, p_ref, st_ref, acc):
    i = pl.program_id(0)
    mA = aggA[...].astype(jnp.float32) / jnp.maximum(cA[:, 0:1], 1.0)
    mB = aggB[...].astype(jnp.float32) / jnp.maximum(cB[:, 0:1], 1.0)
    p = (jnp.dot(mA, wA[...], preferred_element_type=jnp.float32)
         + jnp.dot(mB, wB[...], preferred_element_type=jnp.float32)
         + jnp.dot(x[...], wr[...], preferred_element_type=jnp.float32)
         + bc[...])
    p_ref[...] = p
    s = jnp.sum(p, axis=0, keepdims=True)
    sq = jnp.sum(p * p, axis=0, keepdims=True)

    @pl.when(i == 0)
    def _():
      acc[...] = jnp.zeros_like(acc)

    acc[0:1, :] += s
    acc[1:2, :] += sq

    @pl.when(i == grid - 1)
    def _():
      st_ref[...] = acc[...]

  return pl.pallas_call(
      body,
      grid=(grid,),
      in_specs=[
          pl.BlockSpec((R, _D), lambda i: (i, 0)),
          pl.BlockSpec((R, _D), lambda i: (i, 0)),
          pl.BlockSpec((R, 16), lambda i: (i, 0)),
          pl.BlockSpec((R, 16), lambda i: (i, 0)),
          pl.BlockSpec((_D, _D), lambda i: (0, 0)),
          pl.BlockSpec((_D, _D), lambda i: (0, 0)),
          pl.BlockSpec((_D, _D), lambda i: (0, 0)),
          pl.BlockSpec((1, _D), lambda i: (0, 0)),
          pl.BlockSpec((R, _D), lambda i: (i, 0)),
      ],
      out_specs=[
          pl.BlockSpec((R, _D), lambda i: (i, 0)),
          pl.BlockSpec((8, _D), lambda i: (0, 0)),
      ],
      out_shape=[
          jax.ShapeDtypeStruct((n, _D), jnp.float32),
          jax.ShapeDtypeStruct((8, _D), jnp.float32),
      ],
      scratch_shapes=[pltpu.VMEM((8, _D), jnp.float32)],
      name=name,
  )


def _bn_relu_call(n, name):
  R = 2000
  grid = n // R

  def body(p, st, g, be, o_ref, o16_ref):
    m = st[0:1, :] / float(n)
    var = st[1:2, :] / float(n) - m * m
    scale = g[...] / jnp.sqrt(var + 1e-5)
    v = (p[...] - m) * scale + be[...]
    v = jnp.where(v >= 0, v, v * 0.01)
    o_ref[...] = v
    o16_ref[...] = v.astype(jnp.bfloat16)

  return pl.pallas_call(
      body,
      grid=(grid,),
      in_specs=[
          pl.BlockSpec((R, _D), lambda i: (i, 0)),
          pl.BlockSpec((8, _D), lambda i: (0, 0)),
          pl.BlockSpec((1, _D), lambda i: (0, 0)),
          pl.BlockSpec((1, _D), lambda i: (0, 0)),
      ],
      out_specs=[
          pl.BlockSpec((R, _D), lambda i: (i, 0)),
          pl.BlockSpec((R, _D), lambda i: (i, 0)),
      ],
      out_shape=[
          jax.ShapeDtypeStruct((n, _D), jnp.float32),
          jax.ShapeDtypeStruct((n, _D), jnp.bfloat16),
      ],
      name=name,
  )


def kernel(x_host, x_flow, edge_sends, edge_precedes, edge_rev_sends,
           edge_reaches,
           Wl_0_sends, bl_0_sends, Wr_0_sends,
           Wl_0_precedes, bl_0_precedes, Wr_0_precedes,
           Wl_0_rev_sends, bl_0_rev_sends, Wr_0_rev_sends,
           Wl_0_reaches, bl_0_reaches, Wr_0_reaches,
           g_0, be_0,
           Wl_1_sends, bl_1_sends, Wr_1_sends,
           Wl_1_precedes, bl_1_precedes, Wr_1_precedes,
           Wl_1_rev_sends, bl_1_rev_sends, Wr_1_rev_sends,
           Wl_1_reaches, bl_1_reaches, Wr_1_reaches,
           g_1, be_1):
  def _pack(e):
    return (e[0] | (e[1] << 16)).reshape(_NSUB, _SCAN_ROWS, _LANES)

  e4 = {
      "sends": _pack(edge_sends),
      "precedes": _pack(edge_precedes),
      "rev_sends": _pack(edge_rev_sends),
      "reaches": _pack(edge_reaches),
  }
  W = {
      0: dict(sends=(Wl_0_sends, bl_0_sends, Wr_0_sends),
              precedes=(Wl_0_precedes, bl_0_precedes, Wr_0_precedes),
              rev_sends=(Wl_0_rev_sends, bl_0_rev_sends, Wr_0_rev_sends),
              reaches=(Wl_0_reaches, bl_0_reaches, Wr_0_reaches)),
      1: dict(sends=(Wl_1_sends, bl_1_sends, Wr_1_sends),
              precedes=(Wl_1_precedes, bl_1_precedes, Wr_1_precedes),
              rev_sends=(Wl_1_rev_sends, bl_1_rev_sends, Wr_1_rev_sends),
              reaches=(Wl_1_reaches, bl_1_reaches, Wr_1_reaches)),
  }
  bn = {0: (g_0, be_0), 1: (g_1, be_1)}

  cS, cP, cR, cH = _cntk("cnt_all")(e4["sends"], e4["precedes"],
                                    e4["rev_sends"], e4["reaches"])
  cnt = {"sends": cS, "precedes": cP, "rev_sends": cR, "reaches": cH}

  x = {"host": x_host, "flow": x_flow}
  x16 = {"host": x_host.astype(jnp.bfloat16),
         "flow": x_flow.astype(jnp.bfloat16)}
  for layer in (0, 1):
    aS, aP = _pairk(_CH_FLOW, _N_FLOW, "seg_flow")(
        x16["host"], x16["flow"], e4["sends"], e4["precedes"])
    aR, aH = _pairk(_CH_HOST, _N_HOST, "seg_host")(
        x16["flow"], x16["flow"], e4["rev_sends"], e4["reaches"])
    agg = {"sends": aS, "precedes": aP, "rev_sends": aR, "reaches": aH}

    g, be = bn[layer]
    nxt = {}
    for t, (ra, rb), n in (("flow", ("sends", "precedes"), _N_FLOW),
                           ("host", ("rev_sends", "reaches"), _N_HOST)):
      WlA, blA, WrA = W[layer][ra]
      WlB, blB, WrB = W[layer][rb]
      wrc = 0.5 * (WrA + WrB)
      bc = (0.5 * (blA + blB)).reshape(1, _D)
      comb = _combine_stats_call(n, f"combine_{t}_{layer}")
      p, st = comb(agg[ra], agg[rb], cnt[ra], cnt[rb],
                   0.5 * WlA, 0.5 * WlB, wrc, bc, x[t])
      bnk = _bn_relu_call(n, f"bn_{t}_{layer}")
      nxt[t] = bnk(p, st, g.reshape(1, _D), be.reshape(1, _D))
    x = {t: v[0] for t, v in nxt.items()}
    x16 = {t: v[1] for t, v in nxt.items()}

  return (x["flow"], x["host"])


# speedup vs baseline: 1.7962x; 1.0248x over previous
import functools

import jax
import jax.numpy as jnp
from jax import lax
from jax.experimental import pallas as pl
from jax.experimental.pallas import tpu as pltpu
from jax.experimental.pallas import tpu_sc as plsc

_N_HOST = 10000
_N_FLOW = 50000
_D = 128
_E = 160000

_NCORE = 2
_NSUB = 16
_LANES = 16

_EP = _E // _NSUB
_SCAN_ROWS = _EP // _LANES
_BATCH = 128
_NB_MAX = _EP // _BATCH

_SC_PARAMS = dict(
    compiler_params=pltpu.CompilerParams(needs_layout_passes=False,
                                         use_tc_tiling_on_sc=False))


def _sc_mesh():
  return plsc.VectorSubcoreMesh(core_axis_name="c", subcore_axis_name="s",
                                num_cores=_NCORE, num_subcores=_NSUB)


def _zero_rowbuf(rowbuf):
  z32 = jnp.zeros((2 * _LANES,), jnp.bfloat16)

  def zb(i, _):
    for k in range(_D // (2 * _LANES)):
      rowbuf[i, pl.ds(k * 2 * _LANES, 2 * _LANES)] = z32
    return 0
  lax.fori_loop(0, _BATCH, zb, 0)


def _compact_chunk(ev, dstbuf, srcbuf, lo, ch, dump):
  iota = jnp.arange(_LANES, dtype=jnp.int32)
  zi16 = jnp.zeros((_LANES,), jnp.int32)

  def scan_body(j, posv):
    p16 = ev[j]
    d16 = lax.shift_right_logical(p16, jnp.full((_LANES,), 16, jnp.int32))
    inm = (d16 >= lo) & (d16 < lo + ch)
    ex = plsc.cumsum(inm.astype(jnp.int32))
    tgt = posv + ex - 1
    row = jnp.right_shift(tgt, 7)
    col = jnp.bitwise_and(tgt, _BATCH - 1)
    plsc.store_scatter(dstbuf, [row, col], d16 - lo, mask=inm)
    if srcbuf is not None:
      plsc.store_scatter(srcbuf, [row, col],
                         jnp.bitwise_and(p16, 0xFFFF), mask=inm)
    return posv + plsc.all_reduce_population_count(inm)
  posv = lax.fori_loop(0, _SCAN_ROWS, scan_body, zi16)

  nbv = jnp.right_shift(posv + (_BATCH - 1), 7)
  lastrow = nbv - 1
  for k in range(_BATCH // _LANES):
    colk = k * _LANES + iota
    flatp = lastrow * _BATCH + colk
    m = flatp >= posv
    plsc.store_scatter(dstbuf, [lastrow, colk],
                       jnp.full((_LANES,), dump, jnp.int32), mask=m)
    if srcbuf is not None:
      plsc.store_scatter(srcbuf, [lastrow, colk], zi16, mask=m)
  return jnp.max(nbv)


_CH_FLOW = 12800
_CH_HOST = 5120
_NPAD_FLOW = 4 * _CH_FLOW
_NPAD_HOST = 2 * _CH_HOST
_CHC_FLOW = 25008
_CHC_HOST = 5008


def _agg_relation(x_hbm, e_hbm, agg_hbm, ch, n_dst, refs):
  (ev, srcbuf, dstbuf, bufs, gsems, agg_s) = refs
  cid = lax.axis_index("c")
  sid = lax.axis_index("s")
  nchunk = -(-n_dst // ch)
  assert nchunk % _NCORE == 0 and ch % _NSUB == 0
  dump = ch
  rps = ch // _NSUB
  assert rps % 8 == 0

  pltpu.sync_copy(e_hbm.at[sid], ev)

  for p in range(nchunk // _NCORE):
    chunk = cid + _NCORE * p
    lo = chunk * ch

    _zero_rowbuf(bufs[0])
    rem = rps % _BATCH
    for k in range(rps // _BATCH):
      pltpu.sync_copy(bufs[0], agg_s.at[pl.ds(sid * rps + k * _BATCH,
                                              _BATCH)])
    if rem:
      pltpu.sync_copy(
          bufs[0].at[pl.ds(0, rem)],
          agg_s.at[pl.ds(sid * rps + (rps // _BATCH) * _BATCH, rem)])
    plsc.subcore_barrier()

    nb = _compact_chunk(ev, dstbuf, srcbuf, lo, ch, dump)

    for q in range(3):
      @pl.when(q < nb)
      def _(q=q):
        pltpu.async_copy(x_hbm.at[srcbuf.at[q]], bufs[q], gsems[q])

    def bat(g, _):
      for q in range(3):
        b = 3 * g + q

        @pl.when(b < nb)
        def _(b=b, q=q):
          pltpu.make_async_copy(x_hbm.at[srcbuf.at[b]], bufs[q],
                                gsems[q]).wait()
          pltpu.sync_copy(bufs[q], agg_s.at[dstbuf.at[b]], add=True)

          @pl.when(b + 3 < nb)
          def _():
            pltpu.async_copy(x_hbm.at[srcbuf.at[b + 3]], bufs[q],
                             gsems[q])
      return 0
    lax.fori_loop(0, (_NB_MAX + 2) // 3, bat, 0)

    plsc.subcore_barrier()

    base = lo + sid * rps
    for k in range(rps // _BATCH):
      pltpu.sync_copy(agg_s.at[pl.ds(sid * rps + k * _BATCH, _BATCH)],
                      agg_hbm.at[pl.ds(base + k * _BATCH, _BATCH)])
    if rem:
      pltpu.sync_copy(
          agg_s.at[pl.ds(sid * rps + (rps // _BATCH) * _BATCH, rem)],
          agg_hbm.at[pl.ds(base + (rps // _BATCH) * _BATCH, rem)])
    plsc.subcore_barrier()


def _make_pair_kernel(ch: int, n_dst: int, name: str):
  npad = (-(-n_dst // ch)) * ch
  out_type = (
      jax.ShapeDtypeStruct((npad, _D), jnp.bfloat16),
      jax.ShapeDtypeStruct((npad, _D), jnp.bfloat16),
  )
  scratch = dict(
      ev=pltpu.VMEM((_SCAN_ROWS, _LANES), jnp.int32),
      srcbuf=pltpu.VMEM((_NB_MAX, _BATCH), jnp.int32),
      dstbuf=pltpu.VMEM((_NB_MAX, _BATCH), jnp.int32),
      rowbuf0=pltpu.VMEM((_BATCH, _D), jnp.bfloat16),
      rowbuf1=pltpu.VMEM((_BATCH, _D), jnp.bfloat16),
      rowbuf2=pltpu.VMEM((_BATCH, _D), jnp.bfloat16),
      agg_s=pltpu.VMEM_SHARED((ch + 16, _D), jnp.bfloat16),
      gsem0=pltpu.SemaphoreType.DMA,
      gsem1=pltpu.SemaphoreType.DMA,
      gsem2=pltpu.SemaphoreType.DMA,
  )

  def body(xA_hbm, xB_hbm, eA, eB, aA, aB, *, ev, srcbuf,
           dstbuf, rowbuf0, rowbuf1, rowbuf2, agg_s, gsem0, gsem1, gsem2):
    refs = (ev, srcbuf, dstbuf, (rowbuf0, rowbuf1, rowbuf2),
            (gsem0, gsem1, gsem2), agg_s)
    _agg_relation(xA_hbm, eA, aA, ch, n_dst, refs)
    _agg_relation(xB_hbm, eB, aB, ch, n_dst, refs)

  return pl.kernel(body, out_type=out_type, mesh=_sc_mesh(),
                   scratch_types=scratch, name=name, **_SC_PARAMS)


def _make_cnt_kernel(name: str):
  scratch = dict(
      ev=pltpu.VMEM((_SCAN_ROWS, _LANES), jnp.int32),
      dstbuf=pltpu.VMEM((_NB_MAX, _BATCH), jnp.int32),
      onesb=pltpu.VMEM((_BATCH, 16), jnp.float32),
      zc=pltpu.VMEM((_BATCH, 16), jnp.float32),
      cnt_s=pltpu.VMEM_SHARED((_CHC_FLOW + 16, 16), jnp.float32),
      sem=pltpu.SemaphoreType.DMA,
  )

  def body(eS, eP, eR, eH, cS, cP, cR, cH, *, ev, dstbuf, onesb, zc,
           cnt_s, sem):
    cid = lax.axis_index("c")
    sid = lax.axis_index("s")
    iota = jnp.arange(_LANES, dtype=jnp.int32)
    one0 = (iota == 0).astype(jnp.float32)
    z16 = jnp.zeros((_LANES,), jnp.float32)

    def ob(i, _):
      onesb[i, pl.ds(0, _LANES)] = one0
      zc[i, pl.ds(0, _LANES)] = z16
      return 0
    lax.fori_loop(0, _BATCH, ob, 0)

    for e_hbm, c_hbm, ch in ((eS, cS, _CHC_FLOW), (eP, cP, _CHC_FLOW),
                             (eR, cR, _CHC_HOST), (eH, cH, _CHC_HOST)):
      rps = ch // _NSUB
      dump = ch
      lo = cid * ch
      pltpu.sync_copy(e_hbm.at[sid], ev)

      for k in range(rps // _BATCH):
        pltpu.sync_copy(zc, cnt_s.at[pl.ds(sid * rps + k * _BATCH, _BATCH)])
      rem = rps % _BATCH
      if rem:
        pltpu.sync_copy(
            zc.at[pl.ds(0, rem)],
            cnt_s.at[pl.ds(sid * rps + (rps // _BATCH) * _BATCH, rem)])
      plsc.subcore_barrier()

      nb = _compact_chunk(ev, dstbuf, None, lo, ch, dump)

      def fire(b, _):
        @pl.when(b < nb)
        def _():
          pltpu.async_copy(onesb, cnt_s.at[dstbuf.at[b]], sem, add=True)
        return 0
      lax.fori_loop(0, _NB_MAX, fire, 0)

      def drain(b, _):
        @pl.when(b < nb)
        def _():
          pltpu.make_async_copy(onesb, cnt_s.at[dstbuf.at[b]], sem).wait()
        return 0
      lax.fori_loop(0, _NB_MAX, drain, 0)

      plsc.subcore_barrier()

      base = lo + sid * rps
      pltpu.sync_copy(cnt_s.at[pl.ds(sid * rps, rps)],
                      c_hbm.at[pl.ds(base, rps)])
      plsc.subcore_barrier()

  return pl.kernel(
      body,
      out_type=(jax.ShapeDtypeStruct((_NCORE * _CHC_FLOW, 16), jnp.float32),
                jax.ShapeDtypeStruct((_NCORE * _CHC_FLOW, 16), jnp.float32),
                jax.ShapeDtypeStruct((_NCORE * _CHC_HOST, 16), jnp.float32),
                jax.ShapeDtypeStruct((_NCORE * _CHC_HOST, 16), jnp.float32)),
      mesh=_sc_mesh(), scratch_types=scratch, name=name, **_SC_PARAMS)


@functools.cache
def _pairk(ch, n_dst, name):
  return _make_pair_kernel(ch, n_dst, name)


@functools.cache
def _cntk(name):
  return _make_cnt_kernel(name)


def _combine_bn_call(n, name):
  R = 2000
  nb = n // R

  def rowmap(ph, j):
    return (jnp.where(ph == 0, j, 0), 0)

  def outmap(ph, j):
    return (jnp.where(ph == 1, j, 0), 0)

  def body(aggA, aggB, cA, cB, wA, wB, wr, bc, x, g, be, o_ref, o16_ref,
           pbuf, acc):
    ph = pl.program_id(0)
    j = pl.program_id(1)

    @pl.when(ph == 0)
    def _():
      mA = aggA[...].astype(jnp.float32) / jnp.maximum(cA[:, 0:1], 1.0)
      mB = aggB[...].astype(jnp.float32) / jnp.maximum(cB[:, 0:1], 1.0)
      p = (jnp.dot(mA, wA[...], preferred_element_type=jnp.float32)
           + jnp.dot(mB, wB[...], preferred_element_type=jnp.float32)
           + jnp.dot(x[...], wr[...], preferred_element_type=jnp.float32)
           + bc[...])
      pbuf[pl.ds(j * R, R), :] = p

      @pl.when(j == 0)
      def _():
        acc[...] = jnp.zeros_like(acc)

      acc[0:1, :] += jnp.sum(p, axis=0, keepdims=True)
      acc[1:2, :] += jnp.sum(p * p, axis=0, keepdims=True)

    @pl.when(ph == 1)
    def _():
      m = acc[0:1, :] / float(n)
      var = acc[1:2, :] / float(n) - m * m
      scale = g[...] / jnp.sqrt(var + 1e-5)
      v = (pbuf[pl.ds(j * R, R), :] - m) * scale + be[...]
      v = jnp.where(v >= 0, v, v * 0.01)
      o_ref[...] = v
      o16_ref[...] = v.astype(jnp.bfloat16)

  def cmap(ph, j):
    return (0, 0)

  return pl.pallas_call(
      body,
      grid=(2, nb),
      in_specs=[
          pl.BlockSpec((R, _D), rowmap),
          pl.BlockSpec((R, _D), rowmap),
          pl.BlockSpec((R, 16), rowmap),
          pl.BlockSpec((R, 16), rowmap),
          pl.BlockSpec((_D, _D), cmap),
          pl.BlockSpec((_D, _D), cmap),
          pl.BlockSpec((_D, _D), cmap),
          pl.BlockSpec((1, _D), cmap),
          pl.BlockSpec((R, _D), rowmap),
          pl.BlockSpec((1, _D), cmap),
          pl.BlockSpec((1, _D), cmap),
      ],
      out_specs=[
          pl.BlockSpec((R, _D), outmap),
          pl.BlockSpec((R, _D), outmap),
      ],
      out_shape=[
          jax.ShapeDtypeStruct((n, _D), jnp.float32),
          jax.ShapeDtypeStruct((n, _D), jnp.bfloat16),
      ],
      scratch_shapes=[pltpu.VMEM((n, _D), jnp.float32),
                      pltpu.VMEM((8, _D), jnp.float32)],
      name=name,
  )


def kernel(x_host, x_flow, edge_sends, edge_precedes, edge_rev_sends,
           edge_reaches,
           Wl_0_sends, bl_0_sends, Wr_0_sends,
           Wl_0_precedes, bl_0_precedes, Wr_0_precedes,
           Wl_0_rev_sends, bl_0_rev_sends, Wr_0_rev_sends,
           Wl_0_reaches, bl_0_reaches, Wr_0_reaches,
           g_0, be_0,
           Wl_1_sends, bl_1_sends, Wr_1_sends,
           Wl_1_precedes, bl_1_precedes, Wr_1_precedes,
           Wl_1_rev_sends, bl_1_rev_sends, Wr_1_rev_sends,
           Wl_1_reaches, bl_1_reaches, Wr_1_reaches,
           g_1, be_1):
  def _pack(e):
    return (e[0] | (e[1] << 16)).reshape(_NSUB, _SCAN_ROWS, _LANES)

  e4 = {
      "sends": _pack(edge_sends),
      "precedes": _pack(edge_precedes),
      "rev_sends": _pack(edge_rev_sends),
      "reaches": _pack(edge_reaches),
  }
  W = {
      0: dict(sends=(Wl_0_sends, bl_0_sends, Wr_0_sends),
              precedes=(Wl_0_precedes, bl_0_precedes, Wr_0_precedes),
              rev_sends=(Wl_0_rev_sends, bl_0_rev_sends, Wr_0_rev_sends),
              reaches=(Wl_0_reaches, bl_0_reaches, Wr_0_reaches)),
      1: dict(sends=(Wl_1_sends, bl_1_sends, Wr_1_sends),
              precedes=(Wl_1_precedes, bl_1_precedes, Wr_1_precedes),
              rev_sends=(Wl_1_rev_sends, bl_1_rev_sends, Wr_1_rev_sends),
              reaches=(Wl_1_reaches, bl_1_reaches, Wr_1_reaches)),
  }
  bn = {0: (g_0, be_0), 1: (g_1, be_1)}

  cS, cP, cR, cH = _cntk("cnt_all")(e4["sends"], e4["precedes"],
                                    e4["rev_sends"], e4["reaches"])
  cnt = {"sends": cS, "precedes": cP, "rev_sends": cR, "reaches": cH}

  x = {"host": x_host, "flow": x_flow}
  x16 = {"host": x_host.astype(jnp.bfloat16),
         "flow": x_flow.astype(jnp.bfloat16)}
  for layer in (0, 1):
    aS, aP = _pairk(_CH_FLOW, _N_FLOW, "seg_flow")(
        x16["host"], x16["flow"], e4["sends"], e4["precedes"])
    aR, aH = _pairk(_CH_HOST, _N_HOST, "seg_host")(
        x16["flow"], x16["flow"], e4["rev_sends"], e4["reaches"])
    agg = {"sends": aS, "precedes": aP, "rev_sends": aR, "reaches": aH}

    g, be = bn[layer]
    nxt = {}
    for t, (ra, rb), n in (("flow", ("sends", "precedes"), _N_FLOW),
                           ("host", ("rev_sends", "reaches"), _N_HOST)):
      WlA, blA, WrA = W[layer][ra]
      WlB, blB, WrB = W[layer][rb]
      wrc = 0.5 * (WrA + WrB)
      bc = (0.5 * (blA + blB)).reshape(1, _D)
      comb = _combine_bn_call(n, f"combine_{t}_{layer}")
      nxt[t] = comb(agg[ra], agg[rb], cnt[ra], cnt[rb],
                    0.5 * WlA, 0.5 * WlB, wrc, bc, x[t],
                    g.reshape(1, _D), be.reshape(1, _D))
    x = {t: v[0] for t, v in nxt.items()}
    x16 = {t: v[1] for t, v in nxt.items()}

  return (x["flow"], x["host"])
